# Initial kernel scaffold; baseline (speedup 1.0000x reference)
import functools

import jax
import jax.numpy as jnp
from jax import lax
from jax.experimental import pallas as pl
from jax.experimental.pallas import tpu as pltpu
from jax.experimental.pallas import tpu_sc as plsc

NU, NI, NE, NR = 20000, 30000, 50000, 16
E, NNZ, C, HOPS = 800000, 500000, 64, 2
H = C // 2
CH = 128
ECH = E // CH
NNZP = ((NNZ + CH - 1) // CH) * CH
MCH = NNZP // CH
NIP = NI + 16
NUP = NU + 16
NSC, NTILE, NW = 2, 16, 32

f32 = jnp.float32
i32 = jnp.int32

_mesh = plsc.VectorSubcoreMesh(core_axis_name="c", subcore_axis_name="s")


def _sc_counts(head, mcol, ones16, z16):
    re_, ri_ = NE // NTILE, NIP // NTILE

    @functools.partial(
        pl.kernel, mesh=_mesh,
        out_type=(jax.ShapeDtypeStruct((NSC, NE, 16), f32),
                  jax.ShapeDtypeStruct((NSC, NIP, 16), f32)),
        scratch_types=[pltpu.VMEM((1, CH), i32),
                       pltpu.VMEM((CH, 16), f32),
                       pltpu.VMEM_SHARED((NE, 16), f32),
                       pltpu.VMEM_SHARED((NIP, 16), f32)])
    def k(head_h, mcol_h, ones_h, z_h, ce_h, ci_h, idx_v, ones_v, acc_e, acc_i):
        c = lax.axis_index("c")
        s = lax.axis_index("s")
        w = c * NTILE + s
        pltpu.sync_copy(ones_h, ones_v)
        pltpu.sync_copy(z_h.at[pl.ds(s * re_, re_)], acc_e.at[pl.ds(s * re_, re_)])
        pltpu.sync_copy(z_h.at[pl.ds(s * ri_, ri_)], acc_i.at[pl.ds(s * ri_, ri_)])
        plsc.subcore_barrier()

        n1 = (ECH - w + NW - 1) // NW

        @pl.loop(0, n1)
        def _(i):
            base = (w + i * NW) * CH
            pltpu.sync_copy(head_h.at[pl.ds(base, CH)], idx_v.at[0])
            pltpu.sync_copy(ones_v, acc_e.at[idx_v.at[0]], add=True)

        n2 = (MCH - w + NW - 1) // NW

        @pl.loop(0, n2)
        def _(i):
            base = (w + i * NW) * CH
            pltpu.sync_copy(mcol_h.at[pl.ds(base, CH)], idx_v.at[0])
            pltpu.sync_copy(ones_v, acc_i.at[idx_v.at[0]], add=True)

        plsc.subcore_barrier()
        pltpu.sync_copy(acc_e.at[pl.ds(s * re_, re_)], ce_h.at[c, pl.ds(s * re_, re_)])
        pltpu.sync_copy(acc_i.at[pl.ds(s * ri_, ri_)], ci_h.at[c, pl.ds(s * ri_, ri_)])

    return k(head, mcol, ones16, z16)


def _sc_kg(ent_tbl, tail_aug, etype, head, w_r, z32):
    r = NE // NTILE

    @functools.partial(
        pl.kernel, mesh=_mesh,
        out_type=jax.ShapeDtypeStruct((NSC, NE, H), f32),
        scratch_types=[pltpu.VMEM((1, CH), i32),
                       pltpu.VMEM((1, CH), i32),
                       pltpu.VMEM((1, CH), i32),
                       pltpu.VMEM((CH, H), f32),
                       pltpu.VMEM((CH, H), f32),
                       pltpu.VMEM_SHARED((NE, H), f32),
                       pltpu.VMEM_SHARED((NR, H), f32)])
    def k(ent_h, tail_h, et_h, head_h, w_h, z_h, out_h,
          tidx, ridx, hidx, ent_v, w_v, acc, w_sp):
        c = lax.axis_index("c")
        s = lax.axis_index("s")
        pltpu.sync_copy(z_h.at[pl.ds(s * r, r)], acc.at[pl.ds(s * r, r)])

        @pl.when(s == 0)
        def _():
            pltpu.sync_copy(w_h.at[c], w_v.at[pl.ds(0, NR)])
            pltpu.sync_copy(w_v.at[pl.ds(0, NR)], w_sp)

        plsc.subcore_barrier()

        n = (ECH - s + NTILE - 1) // NTILE

        @pl.loop(0, n)
        def _(i):
            base = (s + i * NTILE) * CH
            pltpu.sync_copy(tail_h.at[c, pl.ds(base, CH)], tidx.at[0])
            pltpu.sync_copy(et_h.at[pl.ds(base, CH)], ridx.at[0])
            pltpu.sync_copy(head_h.at[pl.ds(base, CH)], hidx.at[0])
            pltpu.sync_copy(ent_h.at[tidx.at[0]], ent_v)
            pltpu.sync_copy(w_sp.at[ridx.at[0]], w_v)

            @pl.loop(0, CH)
            def _(j):
                ent_v[j, pl.ds(0, 16)] = ent_v[j, pl.ds(0, 16)] * w_v[j, pl.ds(0, 16)]
                ent_v[j, pl.ds(16, 16)] = ent_v[j, pl.ds(16, 16)] * w_v[j, pl.ds(16, 16)]

            pltpu.sync_copy(ent_v, acc.at[hidx.at[0]], add=True)

        plsc.subcore_barrier()
        pltpu.sync_copy(acc.at[pl.ds(s * r, r)], out_h.at[c, pl.ds(s * r, r)])

    return k(ent_tbl, tail_aug, etype, head, w_r, z32)


def _sc_gs(tbl, gidx_aug, sidx, acc_rows, z32):
    r = acc_rows // NTILE

    @functools.partial(
        pl.kernel, mesh=_mesh,
        out_type=jax.ShapeDtypeStruct((NSC, acc_rows, H), f32),
        scratch_types=[pltpu.VMEM((1, CH), i32),
                       pltpu.VMEM((1, CH), i32),
                       pltpu.VMEM((CH, H), f32),
                       pltpu.VMEM_SHARED((acc_rows, H), f32)])
    def k(tbl_h, g_h, s_h, z_h, out_h, gi, si, rows_v, acc):
        c = lax.axis_index("c")
        s = lax.axis_index("s")
        pltpu.sync_copy(z_h.at[pl.ds(s * r, r)], acc.at[pl.ds(s * r, r)])
        plsc.subcore_barrier()

        n = (MCH - s + NTILE - 1) // NTILE

        @pl.loop(0, n)
        def _(i):
            base = (s + i * NTILE) * CH
            pltpu.sync_copy(g_h.at[c, pl.ds(base, CH)], gi.at[0])
            pltpu.sync_copy(s_h.at[pl.ds(base, CH)], si.at[0])
            pltpu.sync_copy(tbl_h.at[gi.at[0]], rows_v)
            pltpu.sync_copy(rows_v, acc.at[si.at[0]], add=True)

        plsc.subcore_barrier()
        pltpu.sync_copy(acc.at[pl.ds(s * r, r)], out_h.at[c, pl.ds(s * r, r)])

    return k(tbl, gidx_aug, sidx, z32)


def _nrm(x):
    n = jnp.sqrt(jnp.sum(x * x, axis=1, keepdims=True))
    return x / jnp.maximum(n, 1e-12)


BR = 2000
NB_I = NI // BR


def _tc_fuse(kg_sum, cnt_e, int_sum, cnt_i, g1, g2, eres, kres, ires):
    nb = NE // BR

    def body(kg_ref, ce_ref, it_ref, ci_ref, g1_ref, g2_ref, er_ref, kr_ref,
             ir_ref, ent_ref, fus_ref, ero_ref, kro_ref, iro_ref):
        j = pl.program_id(0)
        kg = jnp.concatenate([kg_ref[0], kg_ref[1]], axis=1)
        ce = jnp.maximum(ce_ref[0, :, 0:1] + ce_ref[1, :, 0:1], 1.0)
        kg_agg = kg / ce
        it = jnp.concatenate([it_ref[0], it_ref[1]], axis=1)
        ci = jnp.maximum(ci_ref[0, :, 0:1] + ci_ref[1, :, 0:1], 1.0)
        it_agg = it / ci
        dn = (((1,), (1,)), ((), ()))
        z = (lax.dot_general(kg_agg, g1_ref[...], dn, precision=lax.Precision.HIGHEST)
             + lax.dot_general(it_agg, g2_ref[...], dn, precision=lax.Precision.HIGHEST))
        gi = jax.nn.sigmoid(z)
        fusion = gi * kg_agg + (1.0 - gi) * it_agg
        fus_val = jnp.where(j < NB_I, fusion, kg_agg)
        ent_n = _nrm(fus_val)
        ent_ref[0] = ent_n[:, :H]
        ent_ref[1] = ent_n[:, H:]
        fus_ref[0] = fus_val[:, :H]
        fus_ref[1] = fus_val[:, H:]
        ero_ref[...] = er_ref[...] + ent_n
        kro_ref[...] = kr_ref[...] + _nrm(kg_agg)
        iro_ref[...] = ir_ref[...] + _nrm(it_agg)

    item_blk = lambda j: (0, jnp.minimum(j, NB_I - 1), 0)
    return pl.pallas_call(
        body,
        grid=(nb,),
        in_specs=[
            pl.BlockSpec((NSC, BR, H), lambda j: (0, j, 0)),
            pl.BlockSpec((NSC, BR, 16), lambda j: (0, j, 0)),
            pl.BlockSpec((NSC, BR, H), item_blk),
            pl.BlockSpec((NSC, BR, 16), item_blk),
            pl.BlockSpec((C, C), lambda j: (0, 0)),
            pl.BlockSpec((C, C), lambda j: (0, 0)),
            pl.BlockSpec((BR, C), lambda j: (j, 0)),
            pl.BlockSpec((BR, C), lambda j: (j, 0)),
            pl.BlockSpec((BR, C), lambda j: (j, 0)),
        ],
        out_specs=[
            pl.BlockSpec((NSC, BR, H), lambda j: (0, j, 0)),
            pl.BlockSpec((NSC, BR, H), lambda j: (0, j, 0)),
            pl.BlockSpec((BR, C), lambda j: (j, 0)),
            pl.BlockSpec((BR, C), lambda j: (j, 0)),
            pl.BlockSpec((BR, C), lambda j: (j, 0)),
        ],
        out_shape=[
            jax.ShapeDtypeStruct((NSC, NE, H), f32),
            jax.ShapeDtypeStruct((NSC, NE, H), f32),
            jax.ShapeDtypeStruct((NE, C), f32),
            jax.ShapeDtypeStruct((NE, C), f32),
            jax.ShapeDtypeStruct((NE, C), f32),
        ],
    )(kg_sum, cnt_e, int_sum, cnt_i, g1, g2, eres, kres, ires)


def _tc_user(usum, ures):
    nb = NU // BR

    def body(us_ref, ur_ref, un_ref, uro_ref):
        us = jnp.concatenate([us_ref[0], us_ref[1]], axis=1)
        un = _nrm(us)
        un_ref[0] = un[:, :H]
        un_ref[1] = un[:, H:]
        uro_ref[...] = ur_ref[...] + un

    return pl.pallas_call(
        body,
        grid=(nb,),
        in_specs=[
            pl.BlockSpec((NSC, BR, H), lambda j: (0, j, 0)),
            pl.BlockSpec((BR, C), lambda j: (j, 0)),
        ],
        out_specs=[
            pl.BlockSpec((NSC, BR, H), lambda j: (0, j, 0)),
            pl.BlockSpec((BR, C), lambda j: (j, 0)),
        ],
        out_shape=[
            jax.ShapeDtypeStruct((NSC, NU, H), f32),
            jax.ShapeDtypeStruct((NU, C), f32),
        ],
    )(usum, ures)


def kernel(user_emb, entity_emb, edge_index, edge_type, interact_mat,
           weight, gate1_w, gate2_w):
    head = edge_index[0].astype(i32)
    tail = edge_index[1].astype(i32)
    et = edge_type.astype(i32)
    mrow = interact_mat[0].astype(i32)
    mcol = interact_mat[1].astype(i32)

    pad = NNZP - NNZ
    padr = jnp.arange(pad, dtype=i32) % 16
    tail_aug = jnp.stack([tail, tail + NE])
    mrow_g = jnp.concatenate([mrow, padr])
    mrow_g_aug = jnp.stack([mrow_g, mrow_g + NU])
    mcol_s = jnp.concatenate([mcol, NI + padr])
    mcol_g = jnp.concatenate([mcol, padr])
    mcol_g_aug = jnp.stack([mcol_g, mcol_g + NE])
    mrow_s = jnp.concatenate([mrow, NU + padr])

    z16 = jnp.zeros((NE, 16), f32)
    z32 = jnp.zeros((NE, H), f32)
    ones16 = jnp.ones((CH, 16), f32)

    w_r = weight.reshape(NR, NSC, H).transpose(1, 0, 2)
    ent_tbl = entity_emb.reshape(NE, NSC, H).transpose(1, 0, 2).reshape(NSC * NE, H)
    usr_tbl = user_emb.reshape(NU, NSC, H).transpose(1, 0, 2).reshape(NSC * NU, H)

    cnt_e, cnt_i = _sc_counts(head, mcol_s, ones16, z16)

    eres = entity_emb
    ures = user_emb
    kres = jnp.zeros((NE, C), f32)
    ires = jnp.zeros((NE, C), f32)

    for i in range(HOPS):
        kg_sum = _sc_kg(ent_tbl, tail_aug, et, head, w_r, z32)
        int_sum = _sc_gs(usr_tbl, mrow_g_aug, mcol_s, NIP, z32)
        ent_new, fus, eres, kres, ires = _tc_fuse(
            kg_sum, cnt_e, int_sum, cnt_i, gate1_w[i], gate2_w[i],
            eres, kres, ires)
        usum = _sc_gs(fus.reshape(NSC * NE, H), mcol_g_aug, mrow_s, NUP, z32)
        usr_new, ures = _tc_user(usum, ures)
        ent_tbl = ent_new.reshape(NSC * NE, H)
        usr_tbl = usr_new.reshape(NSC * NU, H)

    return (eres, ures, kres[:NI], ires[:NI])


# speedup vs baseline: 3.1284x; 3.1284x over previous
import functools

import jax
import jax.numpy as jnp
from jax import lax
from jax.experimental import pallas as pl
from jax.experimental.pallas import tpu as pltpu
from jax.experimental.pallas import tpu_sc as plsc

NU, NI, NE, NR = 20000, 30000, 50000, 16
E, NNZ, C, HOPS = 800000, 500000, 64, 2
H = C // 2
CH = 128
ECH = E // CH
NNZP = ((NNZ + CH - 1) // CH) * CH
MCH = NNZP // CH
NEP = 50048
NIP = 30080
NUP = 20096
NSC, NTILE, NW = 2, 16, 32

f32 = jnp.float32
i32 = jnp.int32

_SC_PARAMS = pltpu.CompilerParams(use_tc_tiling_on_sc=False)


@functools.lru_cache(maxsize=None)
def _sc_mesh():
    return plsc.VectorSubcoreMesh(core_axis_name="c", subcore_axis_name="s",
                                  num_cores=NSC, num_subcores=NTILE)


def _sc_counts(head, mcol, ones16, z16):
    re_, ri_ = NEP // NTILE, NIP // NTILE

    @functools.partial(
        pl.kernel, mesh=_sc_mesh(), compiler_params=_SC_PARAMS,
        out_type=(jax.ShapeDtypeStruct((NSC, NEP, 16), f32),
                  jax.ShapeDtypeStruct((NSC, NIP, 16), f32)),
        scratch_types=[pltpu.VMEM((1, CH), i32),
                       pltpu.VMEM((CH, 16), f32),
                       pltpu.VMEM_SHARED((NEP, 16), f32),
                       pltpu.VMEM_SHARED((NIP, 16), f32)])
    def k(head_h, mcol_h, ones_h, z_h, ce_h, ci_h, idx_v, ones_v, acc_e, acc_i):
        c = lax.axis_index("c")
        s = lax.axis_index("s")
        w = c * NTILE + s
        pltpu.sync_copy(ones_h, ones_v)
        pltpu.sync_copy(z_h.at[pl.ds(s * re_, re_)], acc_e.at[pl.ds(s * re_, re_)])
        pltpu.sync_copy(z_h.at[pl.ds(s * ri_, ri_)], acc_i.at[pl.ds(s * ri_, ri_)])
        plsc.subcore_barrier()

        n1 = (ECH - w + NW - 1) // NW

        @pl.loop(0, n1)
        def _(i):
            base = (w + i * NW) * CH
            pltpu.sync_copy(head_h.at[pl.ds(base, CH)], idx_v.at[0])
            pltpu.sync_copy(ones_v, acc_e.at[idx_v.at[0]], add=True)

        n2 = (MCH - w + NW - 1) // NW

        @pl.loop(0, n2)
        def _(i):
            base = (w + i * NW) * CH
            pltpu.sync_copy(mcol_h.at[pl.ds(base, CH)], idx_v.at[0])
            pltpu.sync_copy(ones_v, acc_i.at[idx_v.at[0]], add=True)

        plsc.subcore_barrier()
        pltpu.sync_copy(acc_e.at[pl.ds(s * re_, re_)], ce_h.at[c, pl.ds(s * re_, re_)])
        pltpu.sync_copy(acc_i.at[pl.ds(s * ri_, ri_)], ci_h.at[c, pl.ds(s * ri_, ri_)])

    return k(head, mcol, ones16, z16)


def _sc_kg(ent_tbl, tail_aug, etype, head, w_r, z32):
    r = NEP // NTILE

    @functools.partial(
        pl.kernel, mesh=_sc_mesh(), compiler_params=_SC_PARAMS,
        out_type=jax.ShapeDtypeStruct((NSC, NEP, H), f32),
        scratch_types=[pltpu.VMEM((1, CH), i32),
                       pltpu.VMEM((1, CH), i32),
                       pltpu.VMEM((1, CH), i32),
                       pltpu.VMEM((CH, H), f32),
                       pltpu.VMEM((CH, H), f32),
                       pltpu.VMEM_SHARED((NEP, H), f32),
                       pltpu.VMEM_SHARED((NR, H), f32)])
    def k(ent_h, tail_h, et_h, head_h, w_h, z_h, out_h,
          tidx, ridx, hidx, ent_v, w_v, acc, w_sp):
        c = lax.axis_index("c")
        s = lax.axis_index("s")
        pltpu.sync_copy(z_h.at[pl.ds(s * r, r)], acc.at[pl.ds(s * r, r)])

        @pl.when(s == 0)
        def _():
            pltpu.sync_copy(w_h.at[c], w_v.at[pl.ds(0, NR)])
            pltpu.sync_copy(w_v.at[pl.ds(0, NR)], w_sp)

        plsc.subcore_barrier()

        n = (ECH - s + NTILE - 1) // NTILE

        @pl.loop(0, n)
        def _(i):
            base = (s + i * NTILE) * CH
            pltpu.sync_copy(tail_h.at[c, pl.ds(base, CH)], tidx.at[0])
            pltpu.sync_copy(et_h.at[pl.ds(base, CH)], ridx.at[0])
            pltpu.sync_copy(head_h.at[pl.ds(base, CH)], hidx.at[0])
            pltpu.sync_copy(ent_h.at[tidx.at[0]], ent_v)
            pltpu.sync_copy(w_sp.at[ridx.at[0]], w_v)

            @pl.loop(0, CH)
            def _(j):
                ent_v[j, pl.ds(0, 16)] = ent_v[j, pl.ds(0, 16)] * w_v[j, pl.ds(0, 16)]
                ent_v[j, pl.ds(16, 16)] = ent_v[j, pl.ds(16, 16)] * w_v[j, pl.ds(16, 16)]

            pltpu.sync_copy(ent_v, acc.at[hidx.at[0]], add=True)

        plsc.subcore_barrier()
        pltpu.sync_copy(acc.at[pl.ds(s * r, r)], out_h.at[c, pl.ds(s * r, r)])

    return k(ent_tbl, tail_aug, etype, head, w_r, z32)


def _sc_gs(tbl, gidx_aug, sidx, acc_rows, z32):
    r = acc_rows // NTILE

    @functools.partial(
        pl.kernel, mesh=_sc_mesh(), compiler_params=_SC_PARAMS,
        out_type=jax.ShapeDtypeStruct((NSC, acc_rows, H), f32),
        scratch_types=[pltpu.VMEM((1, CH), i32),
                       pltpu.VMEM((1, CH), i32),
                       pltpu.VMEM((CH, H), f32),
                       pltpu.VMEM_SHARED((acc_rows, H), f32)])
    def k(tbl_h, g_h, s_h, z_h, out_h, gi, si, rows_v, acc):
        c = lax.axis_index("c")
        s = lax.axis_index("s")
        pltpu.sync_copy(z_h.at[pl.ds(s * r, r)], acc.at[pl.ds(s * r, r)])
        plsc.subcore_barrier()

        n = (MCH - s + NTILE - 1) // NTILE

        @pl.loop(0, n)
        def _(i):
            base = (s + i * NTILE) * CH
            pltpu.sync_copy(g_h.at[c, pl.ds(base, CH)], gi.at[0])
            pltpu.sync_copy(s_h.at[pl.ds(base, CH)], si.at[0])
            pltpu.sync_copy(tbl_h.at[gi.at[0]], rows_v)
            pltpu.sync_copy(rows_v, acc.at[si.at[0]], add=True)

        plsc.subcore_barrier()
        pltpu.sync_copy(acc.at[pl.ds(s * r, r)], out_h.at[c, pl.ds(s * r, r)])

    return k(tbl, gidx_aug, sidx, z32)


def _nrm(x):
    n = jnp.sqrt(jnp.sum(x * x, axis=1, keepdims=True))
    return x / jnp.maximum(n, 1e-12)


BR = 2000
NB_I = NI // BR


def _tc_fuse(kg_sum, cnt_e, int_sum, cnt_i, g1, g2, eres, kres, ires):
    nb = NE // BR

    def body(kg_ref, ce_ref, it_ref, ci_ref, g1_ref, g2_ref, er_ref, kr_ref,
             ir_ref, ent_ref, fus_ref, ero_ref, kro_ref, iro_ref):
        j = pl.program_id(0)
        kg = jnp.concatenate([kg_ref[0], kg_ref[1]], axis=1)
        ce = jnp.maximum(ce_ref[0, :, 0:1] + ce_ref[1, :, 0:1], 1.0)
        kg_agg = kg / ce
        it = jnp.concatenate([it_ref[0], it_ref[1]], axis=1)
        ci = jnp.maximum(ci_ref[0, :, 0:1] + ci_ref[1, :, 0:1], 1.0)
        it_agg = it / ci
        dn = (((1,), (1,)), ((), ()))
        z = (lax.dot_general(kg_agg, g1_ref[...], dn, precision=lax.Precision.HIGHEST)
             + lax.dot_general(it_agg, g2_ref[...], dn, precision=lax.Precision.HIGHEST))
        gi = jax.nn.sigmoid(z)
        fusion = gi * kg_agg + (1.0 - gi) * it_agg
        fus_val = jnp.where(j < NB_I, fusion, kg_agg)
        ent_n = _nrm(fus_val)
        ent_ref[0] = ent_n[:, :H]
        ent_ref[1] = ent_n[:, H:]
        fus_ref[0] = fus_val[:, :H]
        fus_ref[1] = fus_val[:, H:]
        ero_ref[...] = er_ref[...] + ent_n
        kro_ref[...] = kr_ref[...] + _nrm(kg_agg)
        iro_ref[...] = ir_ref[...] + _nrm(it_agg)

    item_blk = lambda j: (0, jnp.minimum(j, NB_I - 1), 0)
    return pl.pallas_call(
        body,
        grid=(nb,),
        in_specs=[
            pl.BlockSpec((NSC, BR, H), lambda j: (0, j, 0)),
            pl.BlockSpec((NSC, BR, 16), lambda j: (0, j, 0)),
            pl.BlockSpec((NSC, BR, H), item_blk),
            pl.BlockSpec((NSC, BR, 16), item_blk),
            pl.BlockSpec((C, C), lambda j: (0, 0)),
            pl.BlockSpec((C, C), lambda j: (0, 0)),
            pl.BlockSpec((BR, C), lambda j: (j, 0)),
            pl.BlockSpec((BR, C), lambda j: (j, 0)),
            pl.BlockSpec((BR, C), lambda j: (j, 0)),
        ],
        out_specs=[
            pl.BlockSpec((NSC, BR, H), lambda j: (0, j, 0)),
            pl.BlockSpec((NSC, BR, H), lambda j: (0, j, 0)),
            pl.BlockSpec((BR, C), lambda j: (j, 0)),
            pl.BlockSpec((BR, C), lambda j: (j, 0)),
            pl.BlockSpec((BR, C), lambda j: (j, 0)),
        ],
        out_shape=[
            jax.ShapeDtypeStruct((NSC, NE, H), f32),
            jax.ShapeDtypeStruct((NSC, NE, H), f32),
            jax.ShapeDtypeStruct((NE, C), f32),
            jax.ShapeDtypeStruct((NE, C), f32),
            jax.ShapeDtypeStruct((NE, C), f32),
        ],
    )(kg_sum, cnt_e, int_sum, cnt_i, g1, g2, eres, kres, ires)


def _tc_user(usum, ures):
    nb = NU // BR

    def body(us_ref, ur_ref, un_ref, uro_ref):
        us = jnp.concatenate([us_ref[0], us_ref[1]], axis=1)
        un = _nrm(us)
        un_ref[0] = un[:, :H]
        un_ref[1] = un[:, H:]
        uro_ref[...] = ur_ref[...] + un

    return pl.pallas_call(
        body,
        grid=(nb,),
        in_specs=[
            pl.BlockSpec((NSC, BR, H), lambda j: (0, j, 0)),
            pl.BlockSpec((BR, C), lambda j: (j, 0)),
        ],
        out_specs=[
            pl.BlockSpec((NSC, BR, H), lambda j: (0, j, 0)),
            pl.BlockSpec((BR, C), lambda j: (j, 0)),
        ],
        out_shape=[
            jax.ShapeDtypeStruct((NSC, NU, H), f32),
            jax.ShapeDtypeStruct((NU, C), f32),
        ],
    )(usum, ures)


def kernel(user_emb, entity_emb, edge_index, edge_type, interact_mat,
           weight, gate1_w, gate2_w):
    head = edge_index[0].astype(i32)
    tail = edge_index[1].astype(i32)
    et = edge_type.astype(i32)
    mrow = interact_mat[0].astype(i32)
    mcol = interact_mat[1].astype(i32)

    pad = NNZP - NNZ
    padr = jnp.arange(pad, dtype=i32) % 16
    tail_aug = jnp.stack([tail, tail + NE])
    mrow_g = jnp.concatenate([mrow, padr])
    mrow_g_aug = jnp.stack([mrow_g, mrow_g + NU])
    mcol_s = jnp.concatenate([mcol, NI + padr])
    mcol_g = jnp.concatenate([mcol, padr])
    mcol_g_aug = jnp.stack([mcol_g, mcol_g + NE])
    mrow_s = jnp.concatenate([mrow, NU + padr])

    z16 = jnp.zeros((NEP, 16), f32)
    z32 = jnp.zeros((NEP, H), f32)
    ones16 = jnp.ones((CH, 16), f32)

    w_r = weight.reshape(NR, NSC, H).transpose(1, 0, 2)
    ent_tbl = entity_emb.reshape(NE, NSC, H).transpose(1, 0, 2).reshape(NSC * NE, H)
    usr_tbl = user_emb.reshape(NU, NSC, H).transpose(1, 0, 2).reshape(NSC * NU, H)

    cnt_e, cnt_i = _sc_counts(head, mcol_s, ones16, z16)

    eres = entity_emb
    ures = user_emb
    kres = jnp.zeros((NE, C), f32)
    ires = jnp.zeros((NE, C), f32)

    for i in range(HOPS):
        kg_sum = _sc_kg(ent_tbl, tail_aug, et, head, w_r, z32)
        int_sum = _sc_gs(usr_tbl, mrow_g_aug, mcol_s, NIP, z32)
        ent_new, fus, eres, kres, ires = _tc_fuse(
            kg_sum, cnt_e, int_sum, cnt_i, gate1_w[i], gate2_w[i],
            eres, kres, ires)
        usum = _sc_gs(fus.reshape(NSC * NE, H), mcol_g_aug, mrow_s, NUP, z32)
        usr_new, ures = _tc_user(usum, ures)
        ent_tbl = ent_new.reshape(NSC * NE, H)
        usr_tbl = usr_new.reshape(NSC * NU, H)

    return (eres, ures, kres[:NI], ires[:NI])


# speedup vs baseline: 3.6699x; 1.1731x over previous
import functools

import jax
import jax.numpy as jnp
from jax import lax
from jax.experimental import pallas as pl
from jax.experimental.pallas import tpu as pltpu
from jax.experimental.pallas import tpu_sc as plsc

NU, NI, NE, NR = 20000, 30000, 50000, 16
E, NNZ, C, HOPS = 800000, 500000, 64, 2
H = C // 2
CH = 128
NSC, NTILE, NW = 2, 16, 32

GE = 2
GEDGE = GE * CH
NG_E = 196
EP = NTILE * NG_E * GEDGE
GS = 8
GROW = GS * CH
NG_M = 31
NNZP = NTILE * NG_M * GROW
NG_C = 49

NEP = 50048
NIP = 30080
NUP = 20096

f32 = jnp.float32
i32 = jnp.int32

_SC_PARAMS = pltpu.CompilerParams(use_tc_tiling_on_sc=False)


@functools.lru_cache(maxsize=None)
def _sc_mesh():
    return plsc.VectorSubcoreMesh(core_axis_name="c", subcore_axis_name="s",
                                  num_cores=NSC, num_subcores=NTILE)


def _hist_pipeline(idx_h, ng, ones_v, cidx, acc, ssem, s):
    def load(g, h):
        pltpu.sync_copy(idx_h.at[s * ng + g], cidx.at[h])

    def scat(h, wait):
        if wait:
            return
        for j in range(GS):
            pltpu.sync_copy(ones_v, acc.at[cidx.at[h, j]], add=True)

    load(0, 0)

    @pl.loop(0, ng)
    def _(g):
        p = lax.rem(g, 2)
        q = 1 - p

        @pl.when(g >= 1)
        def _():
            scat(q, wait=True)

        @pl.when(g + 1 < ng)
        def _():
            load(g + 1, q)

        scat(p, wait=False)

    scat(lax.rem(ng - 1, 2), wait=True)


def _sc_counts(head8, col8, ones, z16):
    re_, ri_ = NEP // NTILE, NIP // NTILE

    @functools.partial(
        pl.kernel, mesh=_sc_mesh(), compiler_params=_SC_PARAMS,
        out_type=(jax.ShapeDtypeStruct((NEP, 16), f32),
                  jax.ShapeDtypeStruct((NIP, 16), f32)),
        scratch_types=[pltpu.VMEM((CH, 16), f32),
                       pltpu.VMEM((2, GS, CH), i32),
                       pltpu.VMEM_SHARED((NEP, 16), f32),
                       pltpu.SemaphoreType.DMA])
    def k(head_h, col_h, ones_h, z_h, ce_h, ci_h, ones_v, cidx, acc, ssem):
        c = lax.axis_index("c")
        s = lax.axis_index("s")
        pltpu.sync_copy(ones_h, ones_v)

        @pl.when(c == 0)
        def _():
            pltpu.sync_copy(z_h.at[pl.ds(s * re_, re_)],
                            acc.at[pl.ds(s * re_, re_)])
            plsc.subcore_barrier()
            _hist_pipeline(head_h, NG_C, ones_v, cidx, acc, ssem, s)
            plsc.subcore_barrier()
            pltpu.sync_copy(acc.at[pl.ds(s * re_, re_)],
                            ce_h.at[pl.ds(s * re_, re_)])

        @pl.when(c == 1)
        def _():
            pltpu.sync_copy(z_h.at[pl.ds(s * ri_, ri_)],
                            acc.at[pl.ds(s * ri_, ri_)])
            plsc.subcore_barrier()
            _hist_pipeline(col_h, NG_M, ones_v, cidx, acc, ssem, s)
            plsc.subcore_barrier()
            pltpu.sync_copy(acc.at[pl.ds(s * ri_, ri_)],
                            ci_h.at[pl.ds(s * ri_, ri_)])

    return k(head8, col8, ones, z16)


def _sc_kg(ent_tbl, tail4, et4, head4, w_r, z32):
    r = NEP // NTILE

    @functools.partial(
        pl.kernel, mesh=_sc_mesh(), compiler_params=_SC_PARAMS,
        out_type=jax.ShapeDtypeStruct((NSC, NEP, H), f32),
        scratch_types=[pltpu.VMEM((2, GE, CH), i32),
                       pltpu.VMEM((2, GE, CH), i32),
                       pltpu.VMEM((2, GE, CH), i32),
                       pltpu.VMEM((2, GEDGE, H), f32),
                       pltpu.VMEM((GEDGE, H), f32),
                       pltpu.VMEM_SHARED((NEP, H), f32),
                       pltpu.VMEM_SHARED((NR, H), f32),
                       pltpu.SemaphoreType.DMA,
                       pltpu.SemaphoreType.DMA])
    def k(ent_h, tail_h, et_h, head_h, w_h, z_h, out_h,
          tidx, ridx, hidx, ent_v, w_v, acc, w_sp, gsem, ssem):
        c = lax.axis_index("c")
        s = lax.axis_index("s")
        pltpu.sync_copy(z_h.at[pl.ds(s * r, r)], acc.at[pl.ds(s * r, r)])

        @pl.when(s == 0)
        def _():
            pltpu.sync_copy(w_h.at[c], w_v.at[pl.ds(0, NR)])
            pltpu.sync_copy(w_v.at[pl.ds(0, NR)], w_sp)

        plsc.subcore_barrier()

        def load_and_gather(g, h):
            gb = s * NG_E + g
            pltpu.sync_copy(tail_h.at[c, gb], tidx.at[h])
            pltpu.sync_copy(et_h.at[gb], ridx.at[h])
            pltpu.sync_copy(head_h.at[gb], hidx.at[h])

        def scat(h, wait):
            if wait:
                return
            for j in range(GE):
                pltpu.sync_copy(ent_v.at[h, pl.ds(j * CH, CH)],
                                acc.at[hidx.at[h, j]], add=True)

        load_and_gather(0, 0)

        @pl.loop(0, NG_E)
        def _(g):
            p = lax.rem(g, 2)
            q = 1 - p

            @pl.when(g >= 1)
            def _():
                scat(q, wait=True)

            @pl.when(g + 1 < NG_E)
            def _():
                load_and_gather(g + 1, q)

            for j in range(GE):
                pltpu.sync_copy(ent_h.at[tidx.at[p, j]],
                                ent_v.at[p, pl.ds(j * CH, CH)])
            for j in range(GE):
                pltpu.sync_copy(w_sp.at[ridx.at[p, j]],
                                w_v.at[pl.ds(j * CH, CH)])
            eb = ent_v.at[p]
            wb = w_v

            @pl.loop(0, GEDGE, step=4)
            def _(j):
                for dj in range(4):
                    for h0 in (0, 16):
                        eb[j + dj, pl.ds(h0, 16)] = (
                            eb[j + dj, pl.ds(h0, 16)]
                            * wb[j + dj, pl.ds(h0, 16)])

            scat(p, wait=False)

        scat(lax.rem(NG_E - 1, 2), wait=True)
        plsc.subcore_barrier()
        pltpu.sync_copy(acc.at[pl.ds(s * r, r)], out_h.at[c, pl.ds(s * r, r)])

    return k(ent_tbl, tail4, et4, head4, w_r, z32)


def _sc_gs(tbl, gidx8, sidx8, acc_rows, z32):
    r = acc_rows // NTILE

    @functools.partial(
        pl.kernel, mesh=_sc_mesh(), compiler_params=_SC_PARAMS,
        out_type=jax.ShapeDtypeStruct((NSC, acc_rows, H), f32),
        scratch_types=[pltpu.VMEM((2, GS, CH), i32),
                       pltpu.VMEM((2, GS, CH), i32),
                       pltpu.VMEM((2, GROW, H), f32),
                       pltpu.VMEM_SHARED((acc_rows, H), f32),
                       pltpu.SemaphoreType.DMA,
                       pltpu.SemaphoreType.DMA])
    def k(tbl_h, g_h, s_h, z_h, out_h, gi, si, rows_v, acc, gsem, ssem):
        c = lax.axis_index("c")
        s = lax.axis_index("s")
        pltpu.sync_copy(z_h.at[pl.ds(s * r, r)], acc.at[pl.ds(s * r, r)])
        plsc.subcore_barrier()

        def load_and_gather(g, h):
            gb = s * NG_M + g
            pltpu.sync_copy(g_h.at[c, gb], gi.at[h])
            pltpu.sync_copy(s_h.at[gb], si.at[h])

        def scat(h, wait):
            if wait:
                return
            for j in range(GS):
                pltpu.sync_copy(rows_v.at[h, pl.ds(j * CH, CH)],
                                acc.at[si.at[h, j]], add=True)

        load_and_gather(0, 0)

        @pl.loop(0, NG_M)
        def _(g):
            p = lax.rem(g, 2)
            q = 1 - p

            @pl.when(g >= 1)
            def _():
                scat(q, wait=True)

            @pl.when(g + 1 < NG_M)
            def _():
                load_and_gather(g + 1, q)

            for j in range(GS):
                pltpu.sync_copy(tbl_h.at[gi.at[p, j]],
                                rows_v.at[p, pl.ds(j * CH, CH)])
            scat(p, wait=False)

        scat(lax.rem(NG_M - 1, 2), wait=True)
        plsc.subcore_barrier()
        pltpu.sync_copy(acc.at[pl.ds(s * r, r)], out_h.at[c, pl.ds(s * r, r)])

    return k(tbl, gidx8, sidx8, z32)


def _nrm(x):
    n = jnp.sqrt(jnp.sum(x * x, axis=1, keepdims=True))
    return x / jnp.maximum(n, 1e-12)


BR = 2000
NB_I = NI // BR


def _tc_fuse(kg_sum, cnt_e, int_sum, cnt_i, g1, g2, eres, kres, ires):
    nb = NE // BR

    def body(kg_ref, ce_ref, it_ref, ci_ref, g1_ref, g2_ref, er_ref, kr_ref,
             ir_ref, ent_ref, fus_ref, ero_ref, kro_ref, iro_ref):
        j = pl.program_id(0)
        kg = jnp.concatenate([kg_ref[0], kg_ref[1]], axis=1)
        ce = jnp.maximum(ce_ref[:, 0:1], 1.0)
        kg_agg = kg / ce
        it = jnp.concatenate([it_ref[0], it_ref[1]], axis=1)
        ci = jnp.maximum(ci_ref[:, 0:1], 1.0)
        it_agg = it / ci
        dn = (((1,), (1,)), ((), ()))
        z = (lax.dot_general(kg_agg, g1_ref[...], dn, precision=lax.Precision.HIGHEST)
             + lax.dot_general(it_agg, g2_ref[...], dn, precision=lax.Precision.HIGHEST))
        gi = jax.nn.sigmoid(z)
        fusion = gi * kg_agg + (1.0 - gi) * it_agg
        fus_val = jnp.where(j < NB_I, fusion, kg_agg)
        ent_n = _nrm(fus_val)
        ent_ref[0] = ent_n[:, :H]
        ent_ref[1] = ent_n[:, H:]
        fus_ref[0] = fus_val[:, :H]
        fus_ref[1] = fus_val[:, H:]
        ero_ref[...] = er_ref[...] + ent_n
        kro_ref[...] = kr_ref[...] + _nrm(kg_agg)
        iro_ref[...] = ir_ref[...] + _nrm(it_agg)

    item_blk3 = lambda j: (0, jnp.minimum(j, NB_I - 1), 0)
    item_blk2 = lambda j: (jnp.minimum(j, NB_I - 1), 0)
    return pl.pallas_call(
        body,
        grid=(nb,),
        in_specs=[
            pl.BlockSpec((NSC, BR, H), lambda j: (0, j, 0)),
            pl.BlockSpec((BR, 16), lambda j: (j, 0)),
            pl.BlockSpec((NSC, BR, H), item_blk3),
            pl.BlockSpec((BR, 16), item_blk2),
            pl.BlockSpec((C, C), lambda j: (0, 0)),
            pl.BlockSpec((C, C), lambda j: (0, 0)),
            pl.BlockSpec((BR, C), lambda j: (j, 0)),
            pl.BlockSpec((BR, C), lambda j: (j, 0)),
            pl.BlockSpec((BR, C), lambda j: (j, 0)),
        ],
        out_specs=[
            pl.BlockSpec((NSC, BR, H), lambda j: (0, j, 0)),
            pl.BlockSpec((NSC, BR, H), lambda j: (0, j, 0)),
            pl.BlockSpec((BR, C), lambda j: (j, 0)),
            pl.BlockSpec((BR, C), lambda j: (j, 0)),
            pl.BlockSpec((BR, C), lambda j: (j, 0)),
        ],
        out_shape=[
            jax.ShapeDtypeStruct((NSC, NE, H), f32),
            jax.ShapeDtypeStruct((NSC, NE, H), f32),
            jax.ShapeDtypeStruct((NE, C), f32),
            jax.ShapeDtypeStruct((NE, C), f32),
            jax.ShapeDtypeStruct((NE, C), f32),
        ],
    )(kg_sum, cnt_e, int_sum, cnt_i, g1, g2, eres, kres, ires)


def _tc_user(usum, ures):
    nb = NU // BR

    def body(us_ref, ur_ref, un_ref, uro_ref):
        us = jnp.concatenate([us_ref[0], us_ref[1]], axis=1)
        un = _nrm(us)
        un_ref[0] = un[:, :H]
        un_ref[1] = un[:, H:]
        uro_ref[...] = ur_ref[...] + un

    return pl.pallas_call(
        body,
        grid=(nb,),
        in_specs=[
            pl.BlockSpec((NSC, BR, H), lambda j: (0, j, 0)),
            pl.BlockSpec((BR, C), lambda j: (j, 0)),
        ],
        out_specs=[
            pl.BlockSpec((NSC, BR, H), lambda j: (0, j, 0)),
            pl.BlockSpec((BR, C), lambda j: (j, 0)),
        ],
        out_shape=[
            jax.ShapeDtypeStruct((NSC, NU, H), f32),
            jax.ShapeDtypeStruct((NU, C), f32),
        ],
    )(usum, ures)


def kernel(user_emb, entity_emb, edge_index, edge_type, interact_mat,
           weight, gate1_w, gate2_w):
    head = edge_index[0].astype(i32)
    tail = edge_index[1].astype(i32)
    et = edge_type.astype(i32)
    mrow = interact_mat[0].astype(i32)
    mcol = interact_mat[1].astype(i32)

    epad = EP - E
    eord = jnp.arange(epad, dtype=i32)
    tail_p = jnp.concatenate([tail, eord % 512])
    head_p = jnp.concatenate([head, NE + eord % (NEP - NE)])
    et_p = jnp.concatenate([et, jnp.zeros((epad,), i32)])

    mpad = NNZP - NNZ
    mord = jnp.arange(mpad, dtype=i32)
    mrow_g = jnp.concatenate([mrow, mord % 512])
    mcol_s = jnp.concatenate([mcol, NI + mord % (NIP - NI)])
    mcol_g = jnp.concatenate([mcol, mord % 512])
    mrow_s = jnp.concatenate([mrow, NU + mord % (NUP - NU)])

    tail4 = jnp.stack([tail_p, tail_p + NE]).reshape(NSC, -1, GE, CH)
    et4 = et_p.reshape(-1, GE, CH)
    head4 = head_p.reshape(-1, GE, CH)
    head8 = head_p.reshape(-1, GS, CH)
    mrow_g8 = jnp.stack([mrow_g, mrow_g + NU]).reshape(NSC, -1, GS, CH)
    mcol_s8 = mcol_s.reshape(-1, GS, CH)
    mcol_g8 = jnp.stack([mcol_g, mcol_g + NE]).reshape(NSC, -1, GS, CH)
    mrow_s8 = mrow_s.reshape(-1, GS, CH)

    z16 = jnp.zeros((NEP, 16), f32)
    z32 = jnp.zeros((NEP, H), f32)
    ones = jnp.ones((CH, 16), f32)

    w_r = weight.reshape(NR, NSC, H).transpose(1, 0, 2)
    ent_tbl = entity_emb.reshape(NE, NSC, H).transpose(1, 0, 2).reshape(NSC * NE, H)
    usr_tbl = user_emb.reshape(NU, NSC, H).transpose(1, 0, 2).reshape(NSC * NU, H)

    cnt_e, cnt_i = _sc_counts(head8, mcol_s8, ones, z16)

    eres = entity_emb
    ures = user_emb
    kres = jnp.zeros((NE, C), f32)
    ires = jnp.zeros((NE, C), f32)

    for i in range(HOPS):
        kg_sum = _sc_kg(ent_tbl, tail4, et4, head4, w_r, z32)
        int_sum = _sc_gs(usr_tbl, mrow_g8, mcol_s8, NIP, z32)
        ent_new, fus, eres, kres, ires = _tc_fuse(
            kg_sum, cnt_e, int_sum, cnt_i, gate1_w[i], gate2_w[i],
            eres, kres, ires)
        usum = _sc_gs(fus.reshape(NSC * NE, H), mcol_g8, mrow_s8, NUP, z32)
        usr_new, ures = _tc_user(usum, ures)
        ent_tbl = ent_new.reshape(NSC * NE, H)
        usr_tbl = usr_new.reshape(NSC * NU, H)

    return (eres, ures, kres[:NI], ires[:NI])


# speedup vs baseline: 5.1221x; 1.3957x over previous
import functools

import jax
import jax.numpy as jnp
from jax import lax
from jax.experimental import pallas as pl
from jax.experimental.pallas import tpu as pltpu
from jax.experimental.pallas import tpu_sc as plsc

NU, NI, NE, NR = 20000, 30000, 50000, 16
E, NNZ, C, HOPS = 800000, 500000, 64, 2
H = C // 2
CH = 128
NSC, NTILE, NW = 2, 16, 32

GE = 2
GEDGE = GE * CH
NG_E = 196
EP = NTILE * NG_E * GEDGE
GS = 8
GROW = GS * CH
NG_M = 31
NNZP = NTILE * NG_M * GROW
NG_C = 49

NEP = 50048
NIP = 30080
NUP = 20096

f32 = jnp.float32
i32 = jnp.int32

_SC_PARAMS = pltpu.CompilerParams(use_tc_tiling_on_sc=False)


@functools.lru_cache(maxsize=None)
def _sc_mesh():
    return plsc.VectorSubcoreMesh(core_axis_name="c", subcore_axis_name="s",
                                  num_cores=NSC, num_subcores=NTILE)


def _hist_pipeline(idx_h, ng, ones_v, cidx, acc, ssem, s):
    def load(g, h):
        pltpu.sync_copy(idx_h.at[s * ng + g], cidx.at[h])

    def scat(h, wait):
        if wait:
            return
        for j in range(GS):
            pltpu.sync_copy(ones_v, acc.at[cidx.at[h, j]], add=True)

    load(0, 0)

    @pl.loop(0, ng)
    def _(g):
        p = lax.rem(g, 2)
        q = 1 - p

        @pl.when(g >= 1)
        def _():
            scat(q, wait=True)

        @pl.when(g + 1 < ng)
        def _():
            load(g + 1, q)

        scat(p, wait=False)

    scat(lax.rem(ng - 1, 2), wait=True)


def _sc_counts(head8, col8, ones, z16):
    re_, ri_ = NEP // NTILE, NIP // NTILE

    @functools.partial(
        pl.kernel, mesh=_sc_mesh(), compiler_params=_SC_PARAMS,
        out_type=(jax.ShapeDtypeStruct((NEP, 16), f32),
                  jax.ShapeDtypeStruct((NIP, 16), f32)),
        scratch_types=[pltpu.VMEM((CH, 16), f32),
                       pltpu.VMEM((2, GS, CH), i32),
                       pltpu.VMEM_SHARED((NEP, 16), f32),
                       pltpu.SemaphoreType.DMA])
    def k(head_h, col_h, ones_h, z_h, ce_h, ci_h, ones_v, cidx, acc, ssem):
        c = lax.axis_index("c")
        s = lax.axis_index("s")
        pltpu.sync_copy(ones_h, ones_v)

        @pl.when(c == 0)
        def _():
            pltpu.sync_copy(z_h.at[pl.ds(s * re_, re_)],
                            acc.at[pl.ds(s * re_, re_)])
            plsc.subcore_barrier()
            _hist_pipeline(head_h, NG_C, ones_v, cidx, acc, ssem, s)
            plsc.subcore_barrier()
            pltpu.sync_copy(acc.at[pl.ds(s * re_, re_)],
                            ce_h.at[pl.ds(s * re_, re_)])

        @pl.when(c == 1)
        def _():
            pltpu.sync_copy(z_h.at[pl.ds(s * ri_, ri_)],
                            acc.at[pl.ds(s * ri_, ri_)])
            plsc.subcore_barrier()
            _hist_pipeline(col_h, NG_M, ones_v, cidx, acc, ssem, s)
            plsc.subcore_barrier()
            pltpu.sync_copy(acc.at[pl.ds(s * ri_, ri_)],
                            ci_h.at[pl.ds(s * ri_, ri_)])

    return k(head8, col8, ones, z16)


def _sc_kg(ent_tbl, tail4, et4, head4, w_r, z32):
    r = NEP // NTILE

    @functools.partial(
        pl.kernel, mesh=_sc_mesh(), compiler_params=_SC_PARAMS,
        out_type=jax.ShapeDtypeStruct((NSC, NEP, H), f32),
        scratch_types=[pltpu.VMEM((2, GE, CH), i32),
                       pltpu.VMEM((2, GE, CH), i32),
                       pltpu.VMEM((2, GE, CH), i32),
                       pltpu.VMEM((2, GEDGE, H), f32),
                       pltpu.VMEM((GEDGE, H), f32),
                       pltpu.VMEM_SHARED((NEP, H), f32),
                       pltpu.VMEM_SHARED((NR, H), f32),
                       pltpu.SemaphoreType.DMA,
                       pltpu.SemaphoreType.DMA])
    def k(ent_h, tail_h, et_h, head_h, w_h, z_h, out_h,
          tidx, ridx, hidx, ent_v, w_v, acc, w_sp, gsem, ssem):
        c = lax.axis_index("c")
        s = lax.axis_index("s")
        pltpu.sync_copy(z_h.at[pl.ds(s * r, r)], acc.at[pl.ds(s * r, r)])

        @pl.when(s == 0)
        def _():
            pltpu.sync_copy(w_h.at[c], w_v.at[pl.ds(0, NR)])
            pltpu.sync_copy(w_v.at[pl.ds(0, NR)], w_sp)

        plsc.subcore_barrier()

        def load_and_gather(g, h):
            gb = s * NG_E + g
            pltpu.sync_copy(tail_h.at[c, gb], tidx.at[h])
            pltpu.sync_copy(et_h.at[gb], ridx.at[h])
            pltpu.sync_copy(head_h.at[gb], hidx.at[h])
            for j in range(GE):
                pltpu.async_copy(ent_h.at[tidx.at[h, j]],
                                 ent_v.at[h, pl.ds(j * CH, CH)], gsem)

        def scat(h, wait):
            if wait:
                return
            for j in range(GE):
                pltpu.sync_copy(ent_v.at[h, pl.ds(j * CH, CH)],
                                acc.at[hidx.at[h, j]], add=True)

        load_and_gather(0, 0)

        @pl.loop(0, NG_E)
        def _(g):
            p = lax.rem(g, 2)
            q = 1 - p

            @pl.when(g >= 1)
            def _():
                scat(q, wait=True)

            @pl.when(g + 1 < NG_E)
            def _():
                load_and_gather(g + 1, q)

            for j in range(GE):
                pltpu.make_async_copy(ent_h.at[tidx.at[p, j]],
                                      ent_v.at[p, pl.ds(j * CH, CH)],
                                      gsem).wait()
            for j in range(GE):
                pltpu.sync_copy(w_sp.at[ridx.at[p, j]],
                                w_v.at[pl.ds(j * CH, CH)])
            eb = ent_v.at[p]
            wb = w_v

            @pl.loop(0, GEDGE, step=4)
            def _(j):
                for dj in range(4):
                    for h0 in (0, 16):
                        eb[j + dj, pl.ds(h0, 16)] = (
                            eb[j + dj, pl.ds(h0, 16)]
                            * wb[j + dj, pl.ds(h0, 16)])

            scat(p, wait=False)

        scat(lax.rem(NG_E - 1, 2), wait=True)
        plsc.subcore_barrier()
        pltpu.sync_copy(acc.at[pl.ds(s * r, r)], out_h.at[c, pl.ds(s * r, r)])

    return k(ent_tbl, tail4, et4, head4, w_r, z32)


def _sc_gs(tbl, gidx8, sidx8, acc_rows, z32):
    r = acc_rows // NTILE

    @functools.partial(
        pl.kernel, mesh=_sc_mesh(), compiler_params=_SC_PARAMS,
        out_type=jax.ShapeDtypeStruct((NSC, acc_rows, H), f32),
        scratch_types=[pltpu.VMEM((2, GS, CH), i32),
                       pltpu.VMEM((2, GS, CH), i32),
                       pltpu.VMEM((2, GROW, H), f32),
                       pltpu.VMEM_SHARED((acc_rows, H), f32),
                       pltpu.SemaphoreType.DMA,
                       pltpu.SemaphoreType.DMA])
    def k(tbl_h, g_h, s_h, z_h, out_h, gi, si, rows_v, acc, gsem, ssem):
        c = lax.axis_index("c")
        s = lax.axis_index("s")
        pltpu.sync_copy(z_h.at[pl.ds(s * r, r)], acc.at[pl.ds(s * r, r)])
        plsc.subcore_barrier()

        def load_and_gather(g, h):
            gb = s * NG_M + g
            pltpu.sync_copy(g_h.at[c, gb], gi.at[h])
            pltpu.sync_copy(s_h.at[gb], si.at[h])
            for j in range(GS):
                pltpu.async_copy(tbl_h.at[gi.at[h, j]],
                                 rows_v.at[h, pl.ds(j * CH, CH)], gsem)

        def scat(h, wait):
            if wait:
                return
            for j in range(GS):
                pltpu.sync_copy(rows_v.at[h, pl.ds(j * CH, CH)],
                                acc.at[si.at[h, j]], add=True)

        load_and_gather(0, 0)

        @pl.loop(0, NG_M)
        def _(g):
            p = lax.rem(g, 2)
            q = 1 - p

            @pl.when(g >= 1)
            def _():
                scat(q, wait=True)

            @pl.when(g + 1 < NG_M)
            def _():
                load_and_gather(g + 1, q)

            for j in range(GS):
                pltpu.make_async_copy(tbl_h.at[gi.at[p, j]],
                                      rows_v.at[p, pl.ds(j * CH, CH)],
                                      gsem).wait()
            scat(p, wait=False)

        scat(lax.rem(NG_M - 1, 2), wait=True)
        plsc.subcore_barrier()
        pltpu.sync_copy(acc.at[pl.ds(s * r, r)], out_h.at[c, pl.ds(s * r, r)])

    return k(tbl, gidx8, sidx8, z32)


def _nrm(x):
    n = jnp.sqrt(jnp.sum(x * x, axis=1, keepdims=True))
    return x / jnp.maximum(n, 1e-12)


BR = 2000
NB_I = NI // BR


def _tc_fuse(kg_sum, cnt_e, int_sum, cnt_i, g1, g2, eres, kres, ires):
    nb = NE // BR

    def body(kg_ref, ce_ref, it_ref, ci_ref, g1_ref, g2_ref, er_ref, kr_ref,
             ir_ref, ent_ref, fus_ref, ero_ref, kro_ref, iro_ref):
        j = pl.program_id(0)
        kg = jnp.concatenate([kg_ref[0], kg_ref[1]], axis=1)
        ce = jnp.maximum(ce_ref[:, 0:1], 1.0)
        kg_agg = kg / ce
        it = jnp.concatenate([it_ref[0], it_ref[1]], axis=1)
        ci = jnp.maximum(ci_ref[:, 0:1], 1.0)
        it_agg = it / ci
        dn = (((1,), (1,)), ((), ()))
        z = (lax.dot_general(kg_agg, g1_ref[...], dn, precision=lax.Precision.HIGHEST)
             + lax.dot_general(it_agg, g2_ref[...], dn, precision=lax.Precision.HIGHEST))
        gi = jax.nn.sigmoid(z)
        fusion = gi * kg_agg + (1.0 - gi) * it_agg
        fus_val = jnp.where(j < NB_I, fusion, kg_agg)
        ent_n = _nrm(fus_val)
        ent_ref[0] = ent_n[:, :H]
        ent_ref[1] = ent_n[:, H:]
        fus_ref[0] = fus_val[:, :H]
        fus_ref[1] = fus_val[:, H:]
        ero_ref[...] = er_ref[...] + ent_n
        kro_ref[...] = kr_ref[...] + _nrm(kg_agg)
        iro_ref[...] = ir_ref[...] + _nrm(it_agg)

    item_blk3 = lambda j: (0, jnp.minimum(j, NB_I - 1), 0)
    item_blk2 = lambda j: (jnp.minimum(j, NB_I - 1), 0)
    return pl.pallas_call(
        body,
        grid=(nb,),
        in_specs=[
            pl.BlockSpec((NSC, BR, H), lambda j: (0, j, 0)),
            pl.BlockSpec((BR, 16), lambda j: (j, 0)),
            pl.BlockSpec((NSC, BR, H), item_blk3),
            pl.BlockSpec((BR, 16), item_blk2),
            pl.BlockSpec((C, C), lambda j: (0, 0)),
            pl.BlockSpec((C, C), lambda j: (0, 0)),
            pl.BlockSpec((BR, C), lambda j: (j, 0)),
            pl.BlockSpec((BR, C), lambda j: (j, 0)),
            pl.BlockSpec((BR, C), lambda j: (j, 0)),
        ],
        out_specs=[
            pl.BlockSpec((NSC, BR, H), lambda j: (0, j, 0)),
            pl.BlockSpec((NSC, BR, H), lambda j: (0, j, 0)),
            pl.BlockSpec((BR, C), lambda j: (j, 0)),
            pl.BlockSpec((BR, C), lambda j: (j, 0)),
            pl.BlockSpec((BR, C), lambda j: (j, 0)),
        ],
        out_shape=[
            jax.ShapeDtypeStruct((NSC, NE, H), f32),
            jax.ShapeDtypeStruct((NSC, NE, H), f32),
            jax.ShapeDtypeStruct((NE, C), f32),
            jax.ShapeDtypeStruct((NE, C), f32),
            jax.ShapeDtypeStruct((NE, C), f32),
        ],
    )(kg_sum, cnt_e, int_sum, cnt_i, g1, g2, eres, kres, ires)


def _tc_user(usum, ures):
    nb = NU // BR

    def body(us_ref, ur_ref, un_ref, uro_ref):
        us = jnp.concatenate([us_ref[0], us_ref[1]], axis=1)
        un = _nrm(us)
        un_ref[0] = un[:, :H]
        un_ref[1] = un[:, H:]
        uro_ref[...] = ur_ref[...] + un

    return pl.pallas_call(
        body,
        grid=(nb,),
        in_specs=[
            pl.BlockSpec((NSC, BR, H), lambda j: (0, j, 0)),
            pl.BlockSpec((BR, C), lambda j: (j, 0)),
        ],
        out_specs=[
            pl.BlockSpec((NSC, BR, H), lambda j: (0, j, 0)),
            pl.BlockSpec((BR, C), lambda j: (j, 0)),
        ],
        out_shape=[
            jax.ShapeDtypeStruct((NSC, NU, H), f32),
            jax.ShapeDtypeStruct((NU, C), f32),
        ],
    )(usum, ures)


def kernel(user_emb, entity_emb, edge_index, edge_type, interact_mat,
           weight, gate1_w, gate2_w):
    head = edge_index[0].astype(i32)
    tail = edge_index[1].astype(i32)
    et = edge_type.astype(i32)
    mrow = interact_mat[0].astype(i32)
    mcol = interact_mat[1].astype(i32)

    epad = EP - E
    eord = jnp.arange(epad, dtype=i32)
    tail_p = jnp.concatenate([tail, eord % 512])
    head_p = jnp.concatenate([head, NE + eord % (NEP - NE)])
    et_p = jnp.concatenate([et, jnp.zeros((epad,), i32)])

    mpad = NNZP - NNZ
    mord = jnp.arange(mpad, dtype=i32)
    mrow_g = jnp.concatenate([mrow, mord % 512])
    mcol_s = jnp.concatenate([mcol, NI + mord % (NIP - NI)])
    mcol_g = jnp.concatenate([mcol, mord % 512])
    mrow_s = jnp.concatenate([mrow, NU + mord % (NUP - NU)])

    tail4 = jnp.stack([tail_p, tail_p + NE]).reshape(NSC, -1, GE, CH)
    et4 = et_p.reshape(-1, GE, CH)
    head4 = head_p.reshape(-1, GE, CH)
    head8 = head_p.reshape(-1, GS, CH)
    mrow_g8 = jnp.stack([mrow_g, mrow_g + NU]).reshape(NSC, -1, GS, CH)
    mcol_s8 = mcol_s.reshape(-1, GS, CH)
    mcol_g8 = jnp.stack([mcol_g, mcol_g + NE]).reshape(NSC, -1, GS, CH)
    mrow_s8 = mrow_s.reshape(-1, GS, CH)

    z16 = jnp.zeros((NEP, 16), f32)
    z32 = jnp.zeros((NEP, H), f32)
    ones = jnp.ones((CH, 16), f32)

    w_r = weight.reshape(NR, NSC, H).transpose(1, 0, 2)
    ent_tbl = entity_emb.reshape(NE, NSC, H).transpose(1, 0, 2).reshape(NSC * NE, H)
    usr_tbl = user_emb.reshape(NU, NSC, H).transpose(1, 0, 2).reshape(NSC * NU, H)

    cnt_e, cnt_i = _sc_counts(head8, mcol_s8, ones, z16)

    eres = entity_emb
    ures = user_emb
    kres = jnp.zeros((NE, C), f32)
    ires = jnp.zeros((NE, C), f32)

    for i in range(HOPS):
        kg_sum = _sc_kg(ent_tbl, tail4, et4, head4, w_r, z32)
        int_sum = _sc_gs(usr_tbl, mrow_g8, mcol_s8, NIP, z32)
        ent_new, fus, eres, kres, ires = _tc_fuse(
            kg_sum, cnt_e, int_sum, cnt_i, gate1_w[i], gate2_w[i],
            eres, kres, ires)
        usum = _sc_gs(fus.reshape(NSC * NE, H), mcol_g8, mrow_s8, NUP, z32)
        usr_new, ures = _tc_user(usum, ures)
        ent_tbl = ent_new.reshape(NSC * NE, H)
        usr_tbl = usr_new.reshape(NSC * NU, H)

    return (eres, ures, kres[:NI], ires[:NI])


# speedup vs baseline: 5.2235x; 1.0198x over previous
import functools

import jax
import jax.numpy as jnp
from jax import lax
from jax.experimental import pallas as pl
from jax.experimental.pallas import tpu as pltpu
from jax.experimental.pallas import tpu_sc as plsc

NU, NI, NE, NR = 20000, 30000, 50000, 16
E, NNZ, C, HOPS = 800000, 500000, 64, 2
H = C // 2
CH = 128
NSC, NTILE, NW = 2, 16, 32

GE = 2
GEDGE = GE * CH
NG_E = 196
EP = NTILE * NG_E * GEDGE
GS = 8
GROW = GS * CH
NG_M = 31
NNZP = NTILE * NG_M * GROW
NG_C = 49

NEP = 50048
NIP = 30080
NUP = 20096

f32 = jnp.float32
i32 = jnp.int32

_SC_PARAMS = pltpu.CompilerParams(use_tc_tiling_on_sc=False)


@functools.lru_cache(maxsize=None)
def _sc_mesh():
    return plsc.VectorSubcoreMesh(core_axis_name="c", subcore_axis_name="s",
                                  num_cores=NSC, num_subcores=NTILE)


def _hist_pipeline(idx_h, ng, ones_v, cidx, acc, ssem, s):
    def load(g, h):
        pltpu.sync_copy(idx_h.at[s * ng + g], cidx.at[h])

    def scat(h, wait):
        if wait:
            return
        hs = [pltpu.async_copy(ones_v, acc.at[cidx.at[h, j]], ssem, add=True)
              for j in range(GS)]
        for r_ in hs:
            r_.wait()

    load(0, 0)

    @pl.loop(0, ng)
    def _(g):
        p = lax.rem(g, 2)
        q = 1 - p

        @pl.when(g >= 1)
        def _():
            scat(q, wait=True)

        @pl.when(g + 1 < ng)
        def _():
            load(g + 1, q)

        scat(p, wait=False)

    scat(lax.rem(ng - 1, 2), wait=True)


def _sc_counts(head8, col8, ones, z16):
    re_, ri_ = NEP // NTILE, NIP // NTILE

    @functools.partial(
        pl.kernel, mesh=_sc_mesh(), compiler_params=_SC_PARAMS,
        out_type=(jax.ShapeDtypeStruct((NEP, 16), f32),
                  jax.ShapeDtypeStruct((NIP, 16), f32)),
        scratch_types=[pltpu.VMEM((CH, 16), f32),
                       pltpu.VMEM((2, GS, CH), i32),
                       pltpu.VMEM_SHARED((NEP, 16), f32),
                       pltpu.SemaphoreType.DMA])
    def k(head_h, col_h, ones_h, z_h, ce_h, ci_h, ones_v, cidx, acc, ssem):
        c = lax.axis_index("c")
        s = lax.axis_index("s")
        pltpu.sync_copy(ones_h, ones_v)

        @pl.when(c == 0)
        def _():
            pltpu.sync_copy(z_h.at[pl.ds(s * re_, re_)],
                            acc.at[pl.ds(s * re_, re_)])
            plsc.subcore_barrier()
            _hist_pipeline(head_h, NG_C, ones_v, cidx, acc, ssem, s)
            plsc.subcore_barrier()
            pltpu.sync_copy(acc.at[pl.ds(s * re_, re_)],
                            ce_h.at[pl.ds(s * re_, re_)])

        @pl.when(c == 1)
        def _():
            pltpu.sync_copy(z_h.at[pl.ds(s * ri_, ri_)],
                            acc.at[pl.ds(s * ri_, ri_)])
            plsc.subcore_barrier()
            _hist_pipeline(col_h, NG_M, ones_v, cidx, acc, ssem, s)
            plsc.subcore_barrier()
            pltpu.sync_copy(acc.at[pl.ds(s * ri_, ri_)],
                            ci_h.at[pl.ds(s * ri_, ri_)])

    return k(head8, col8, ones, z16)


def _sc_kg(ent_tbl, tail4, et4, head4, w_r, z32):
    r = NEP // NTILE

    @functools.partial(
        pl.kernel, mesh=_sc_mesh(), compiler_params=_SC_PARAMS,
        out_type=jax.ShapeDtypeStruct((NSC, NEP, H), f32),
        scratch_types=[pltpu.VMEM((2, GE, CH), i32),
                       pltpu.VMEM((2, GE, CH), i32),
                       pltpu.VMEM((2, GE, CH), i32),
                       pltpu.VMEM((2, GEDGE, H), f32),
                       pltpu.VMEM((GEDGE, H), f32),
                       pltpu.VMEM_SHARED((NEP, H), f32),
                       pltpu.VMEM_SHARED((NR, H), f32),
                       pltpu.SemaphoreType.DMA,
                       pltpu.SemaphoreType.DMA])
    def k(ent_h, tail_h, et_h, head_h, w_h, z_h, out_h,
          tidx, ridx, hidx, ent_v, w_v, acc, w_sp, gsem, ssem):
        c = lax.axis_index("c")
        s = lax.axis_index("s")
        pltpu.sync_copy(z_h.at[pl.ds(s * r, r)], acc.at[pl.ds(s * r, r)])

        @pl.when(s == 0)
        def _():
            pltpu.sync_copy(w_h.at[c], w_v.at[pl.ds(0, NR)])
            pltpu.sync_copy(w_v.at[pl.ds(0, NR)], w_sp)

        plsc.subcore_barrier()

        def load_and_gather(g, h):
            gb = s * NG_E + g
            pltpu.sync_copy(tail_h.at[c, gb], tidx.at[h])
            pltpu.sync_copy(et_h.at[gb], ridx.at[h])
            pltpu.sync_copy(head_h.at[gb], hidx.at[h])
            for j in range(GE):
                pltpu.async_copy(ent_h.at[tidx.at[h, j]],
                                 ent_v.at[h, pl.ds(j * CH, CH)], gsem)

        def scat(h, wait):
            if wait:
                return
            hs = [pltpu.async_copy(ent_v.at[h, pl.ds(j * CH, CH)],
                                   acc.at[hidx.at[h, j]], ssem, add=True)
                  for j in range(GE)]
            for r_ in hs:
                r_.wait()

        load_and_gather(0, 0)

        @pl.loop(0, NG_E)
        def _(g):
            p = lax.rem(g, 2)
            q = 1 - p

            @pl.when(g >= 1)
            def _():
                scat(q, wait=True)

            @pl.when(g + 1 < NG_E)
            def _():
                load_and_gather(g + 1, q)

            for j in range(GE):
                pltpu.make_async_copy(ent_h.at[tidx.at[p, j]],
                                      ent_v.at[p, pl.ds(j * CH, CH)],
                                      gsem).wait()
            for j in range(GE):
                pltpu.sync_copy(w_sp.at[ridx.at[p, j]],
                                w_v.at[pl.ds(j * CH, CH)])
            eb = ent_v.at[p]
            wb = w_v

            @pl.loop(0, GEDGE, step=4)
            def _(j):
                for dj in range(4):
                    for h0 in (0, 16):
                        eb[j + dj, pl.ds(h0, 16)] = (
                            eb[j + dj, pl.ds(h0, 16)]
                            * wb[j + dj, pl.ds(h0, 16)])

            scat(p, wait=False)

        scat(lax.rem(NG_E - 1, 2), wait=True)
        plsc.subcore_barrier()
        pltpu.sync_copy(acc.at[pl.ds(s * r, r)], out_h.at[c, pl.ds(s * r, r)])

    return k(ent_tbl, tail4, et4, head4, w_r, z32)


def _sc_gs(tbl, gidx8, sidx8, acc_rows, z32):
    r = acc_rows // NTILE

    @functools.partial(
        pl.kernel, mesh=_sc_mesh(), compiler_params=_SC_PARAMS,
        out_type=jax.ShapeDtypeStruct((NSC, acc_rows, H), f32),
        scratch_types=[pltpu.VMEM((2, GS, CH), i32),
                       pltpu.VMEM((2, GS, CH), i32),
                       pltpu.VMEM((2, GROW, H), f32),
                       pltpu.VMEM_SHARED((acc_rows, H), f32),
                       pltpu.SemaphoreType.DMA,
                       pltpu.SemaphoreType.DMA])
    def k(tbl_h, g_h, s_h, z_h, out_h, gi, si, rows_v, acc, gsem, ssem):
        c = lax.axis_index("c")
        s = lax.axis_index("s")
        pltpu.sync_copy(z_h.at[pl.ds(s * r, r)], acc.at[pl.ds(s * r, r)])
        plsc.subcore_barrier()

        def load_and_gather(g, h):
            gb = s * NG_M + g
            pltpu.sync_copy(g_h.at[c, gb], gi.at[h])
            pltpu.sync_copy(s_h.at[gb], si.at[h])
            for j in range(GS):
                pltpu.async_copy(tbl_h.at[gi.at[h, j]],
                                 rows_v.at[h, pl.ds(j * CH, CH)], gsem)

        def scat(h, wait):
            if wait:
                return
            hs = [pltpu.async_copy(rows_v.at[h, pl.ds(j * CH, CH)],
                                   acc.at[si.at[h, j]], ssem, add=True)
                  for j in range(GS)]
            for r_ in hs:
                r_.wait()

        load_and_gather(0, 0)

        @pl.loop(0, NG_M)
        def _(g):
            p = lax.rem(g, 2)
            q = 1 - p

            @pl.when(g >= 1)
            def _():
                scat(q, wait=True)

            @pl.when(g + 1 < NG_M)
            def _():
                load_and_gather(g + 1, q)

            for j in range(GS):
                pltpu.make_async_copy(tbl_h.at[gi.at[p, j]],
                                      rows_v.at[p, pl.ds(j * CH, CH)],
                                      gsem).wait()
            scat(p, wait=False)

        scat(lax.rem(NG_M - 1, 2), wait=True)
        plsc.subcore_barrier()
        pltpu.sync_copy(acc.at[pl.ds(s * r, r)], out_h.at[c, pl.ds(s * r, r)])

    return k(tbl, gidx8, sidx8, z32)


def _nrm(x):
    n = jnp.sqrt(jnp.sum(x * x, axis=1, keepdims=True))
    return x / jnp.maximum(n, 1e-12)


BR = 2000
NB_I = NI // BR


def _tc_fuse(kg_sum, cnt_e, int_sum, cnt_i, g1, g2, eres, kres, ires):
    nb = NE // BR

    def body(kg_ref, ce_ref, it_ref, ci_ref, g1_ref, g2_ref, er_ref, kr_ref,
             ir_ref, ent_ref, fus_ref, ero_ref, kro_ref, iro_ref):
        j = pl.program_id(0)
        kg = jnp.concatenate([kg_ref[0], kg_ref[1]], axis=1)
        ce = jnp.maximum(ce_ref[:, 0:1], 1.0)
        kg_agg = kg / ce
        it = jnp.concatenate([it_ref[0], it_ref[1]], axis=1)
        ci = jnp.maximum(ci_ref[:, 0:1], 1.0)
        it_agg = it / ci
        dn = (((1,), (1,)), ((), ()))
        z = (lax.dot_general(kg_agg, g1_ref[...], dn, precision=lax.Precision.HIGHEST)
             + lax.dot_general(it_agg, g2_ref[...], dn, precision=lax.Precision.HIGHEST))
        gi = jax.nn.sigmoid(z)
        fusion = gi * kg_agg + (1.0 - gi) * it_agg
        fus_val = jnp.where(j < NB_I, fusion, kg_agg)
        ent_n = _nrm(fus_val)
        ent_ref[0] = ent_n[:, :H]
        ent_ref[1] = ent_n[:, H:]
        fus_ref[0] = fus_val[:, :H]
        fus_ref[1] = fus_val[:, H:]
        ero_ref[...] = er_ref[...] + ent_n
        kro_ref[...] = kr_ref[...] + _nrm(kg_agg)
        iro_ref[...] = ir_ref[...] + _nrm(it_agg)

    item_blk3 = lambda j: (0, jnp.minimum(j, NB_I - 1), 0)
    item_blk2 = lambda j: (jnp.minimum(j, NB_I - 1), 0)
    return pl.pallas_call(
        body,
        grid=(nb,),
        in_specs=[
            pl.BlockSpec((NSC, BR, H), lambda j: (0, j, 0)),
            pl.BlockSpec((BR, 16), lambda j: (j, 0)),
            pl.BlockSpec((NSC, BR, H), item_blk3),
            pl.BlockSpec((BR, 16), item_blk2),
            pl.BlockSpec((C, C), lambda j: (0, 0)),
            pl.BlockSpec((C, C), lambda j: (0, 0)),
            pl.BlockSpec((BR, C), lambda j: (j, 0)),
            pl.BlockSpec((BR, C), lambda j: (j, 0)),
            pl.BlockSpec((BR, C), lambda j: (j, 0)),
        ],
        out_specs=[
            pl.BlockSpec((NSC, BR, H), lambda j: (0, j, 0)),
            pl.BlockSpec((NSC, BR, H), lambda j: (0, j, 0)),
            pl.BlockSpec((BR, C), lambda j: (j, 0)),
            pl.BlockSpec((BR, C), lambda j: (j, 0)),
            pl.BlockSpec((BR, C), lambda j: (j, 0)),
        ],
        out_shape=[
            jax.ShapeDtypeStruct((NSC, NE, H), f32),
            jax.ShapeDtypeStruct((NSC, NE, H), f32),
            jax.ShapeDtypeStruct((NE, C), f32),
            jax.ShapeDtypeStruct((NE, C), f32),
            jax.ShapeDtypeStruct((NE, C), f32),
        ],
    )(kg_sum, cnt_e, int_sum, cnt_i, g1, g2, eres, kres, ires)


def _tc_user(usum, ures):
    nb = NU // BR

    def body(us_ref, ur_ref, un_ref, uro_ref):
        us = jnp.concatenate([us_ref[0], us_ref[1]], axis=1)
        un = _nrm(us)
        un_ref[0] = un[:, :H]
        un_ref[1] = un[:, H:]
        uro_ref[...] = ur_ref[...] + un

    return pl.pallas_call(
        body,
        grid=(nb,),
        in_specs=[
            pl.BlockSpec((NSC, BR, H), lambda j: (0, j, 0)),
            pl.BlockSpec((BR, C), lambda j: (j, 0)),
        ],
        out_specs=[
            pl.BlockSpec((NSC, BR, H), lambda j: (0, j, 0)),
            pl.BlockSpec((BR, C), lambda j: (j, 0)),
        ],
        out_shape=[
            jax.ShapeDtypeStruct((NSC, NU, H), f32),
            jax.ShapeDtypeStruct((NU, C), f32),
        ],
    )(usum, ures)


def kernel(user_emb, entity_emb, edge_index, edge_type, interact_mat,
           weight, gate1_w, gate2_w):
    head = edge_index[0].astype(i32)
    tail = edge_index[1].astype(i32)
    et = edge_type.astype(i32)
    mrow = interact_mat[0].astype(i32)
    mcol = interact_mat[1].astype(i32)

    epad = EP - E
    eord = jnp.arange(epad, dtype=i32)
    tail_p = jnp.concatenate([tail, eord % 512])
    head_p = jnp.concatenate([head, NE + eord % (NEP - NE)])
    et_p = jnp.concatenate([et, jnp.zeros((epad,), i32)])

    mpad = NNZP - NNZ
    mord = jnp.arange(mpad, dtype=i32)
    mrow_g = jnp.concatenate([mrow, mord % 512])
    mcol_s = jnp.concatenate([mcol, NI + mord % (NIP - NI)])
    mcol_g = jnp.concatenate([mcol, mord % 512])
    mrow_s = jnp.concatenate([mrow, NU + mord % (NUP - NU)])

    tail4 = jnp.stack([tail_p, tail_p + NE]).reshape(NSC, -1, GE, CH)
    et4 = et_p.reshape(-1, GE, CH)
    head4 = head_p.reshape(-1, GE, CH)
    head8 = head_p.reshape(-1, GS, CH)
    mrow_g8 = jnp.stack([mrow_g, mrow_g + NU]).reshape(NSC, -1, GS, CH)
    mcol_s8 = mcol_s.reshape(-1, GS, CH)
    mcol_g8 = jnp.stack([mcol_g, mcol_g + NE]).reshape(NSC, -1, GS, CH)
    mrow_s8 = mrow_s.reshape(-1, GS, CH)

    z16 = jnp.zeros((NEP, 16), f32)
    z32 = jnp.zeros((NEP, H), f32)
    ones = jnp.ones((CH, 16), f32)

    w_r = weight.reshape(NR, NSC, H).transpose(1, 0, 2)
    ent_tbl = entity_emb.reshape(NE, NSC, H).transpose(1, 0, 2).reshape(NSC * NE, H)
    usr_tbl = user_emb.reshape(NU, NSC, H).transpose(1, 0, 2).reshape(NSC * NU, H)

    cnt_e, cnt_i = _sc_counts(head8, mcol_s8, ones, z16)

    eres = entity_emb
    ures = user_emb
    kres = jnp.zeros((NE, C), f32)
    ires = jnp.zeros((NE, C), f32)

    for i in range(HOPS):
        kg_sum = _sc_kg(ent_tbl, tail4, et4, head4, w_r, z32)
        int_sum = _sc_gs(usr_tbl, mrow_g8, mcol_s8, NIP, z32)
        ent_new, fus, eres, kres, ires = _tc_fuse(
            kg_sum, cnt_e, int_sum, cnt_i, gate1_w[i], gate2_w[i],
            eres, kres, ires)
        usum = _sc_gs(fus.reshape(NSC * NE, H), mcol_g8, mrow_s8, NUP, z32)
        usr_new, ures = _tc_user(usum, ures)
        ent_tbl = ent_new.reshape(NSC * NE, H)
        usr_tbl = usr_new.reshape(NSC * NU, H)

    return (eres, ures, kres[:NI], ires[:NI])


# speedup vs baseline: 5.9073x; 1.1309x over previous
import functools

import jax
import jax.numpy as jnp
from jax import lax
from jax.experimental import pallas as pl
from jax.experimental.pallas import tpu as pltpu
from jax.experimental.pallas import tpu_sc as plsc

NU, NI, NE, NR = 20000, 30000, 50000, 16
E, NNZ, C, HOPS = 800000, 500000, 64, 2
H = C // 2
CH = 128
NSC, NTILE, NW = 2, 16, 32

GE = 2
GEDGE = GE * CH
NG_E = 196
EP = NTILE * NG_E * GEDGE
GS = 8
GROW = GS * CH
NG_M = 31
NNZP = NTILE * NG_M * GROW
NG_C = 49

NEP = 50048
NIP = 30080
NUP = 20096

f32 = jnp.float32
i32 = jnp.int32

_SC_PARAMS = pltpu.CompilerParams(use_tc_tiling_on_sc=False)


@functools.lru_cache(maxsize=None)
def _sc_mesh():
    return plsc.VectorSubcoreMesh(core_axis_name="c", subcore_axis_name="s",
                                  num_cores=NSC, num_subcores=NTILE)


def _hist_pipeline(idx_h, ng, ones_v, cidx, acc, ssem, s):
    def load(g, h):
        pltpu.sync_copy(idx_h.at[s * ng + g], cidx.at[h])

    def scat(h, wait):
        if wait:
            return
        hs = [pltpu.async_copy(ones_v, acc.at[cidx.at[h, j]], ssem, add=True)
              for j in range(GS)]
        for r_ in hs:
            r_.wait()

    load(0, 0)

    @pl.loop(0, ng)
    def _(g):
        p = lax.rem(g, 2)
        q = 1 - p

        @pl.when(g >= 1)
        def _():
            scat(q, wait=True)

        @pl.when(g + 1 < ng)
        def _():
            load(g + 1, q)

        scat(p, wait=False)

    scat(lax.rem(ng - 1, 2), wait=True)


def _sc_counts(head8, col8, ones, z16):
    re_, ri_ = NEP // NTILE, NIP // NTILE

    @functools.partial(
        pl.kernel, mesh=_sc_mesh(), compiler_params=_SC_PARAMS,
        out_type=(jax.ShapeDtypeStruct((NEP, 16), f32),
                  jax.ShapeDtypeStruct((NIP, 16), f32)),
        scratch_types=[pltpu.VMEM((CH, 16), f32),
                       pltpu.VMEM((2, GS, CH), i32),
                       pltpu.VMEM_SHARED((NEP, 16), f32),
                       pltpu.SemaphoreType.DMA])
    def k(head_h, col_h, ones_h, z_h, ce_h, ci_h, ones_v, cidx, acc, ssem):
        c = lax.axis_index("c")
        s = lax.axis_index("s")
        pltpu.sync_copy(ones_h, ones_v)

        @pl.when(c == 0)
        def _():
            pltpu.sync_copy(z_h.at[pl.ds(s * re_, re_)],
                            acc.at[pl.ds(s * re_, re_)])
            plsc.subcore_barrier()
            _hist_pipeline(head_h, NG_C, ones_v, cidx, acc, ssem, s)
            plsc.subcore_barrier()
            pltpu.sync_copy(acc.at[pl.ds(s * re_, re_)],
                            ce_h.at[pl.ds(s * re_, re_)])

        @pl.when(c == 1)
        def _():
            pltpu.sync_copy(z_h.at[pl.ds(s * ri_, ri_)],
                            acc.at[pl.ds(s * ri_, ri_)])
            plsc.subcore_barrier()
            _hist_pipeline(col_h, NG_M, ones_v, cidx, acc, ssem, s)
            plsc.subcore_barrier()
            pltpu.sync_copy(acc.at[pl.ds(s * ri_, ri_)],
                            ci_h.at[pl.ds(s * ri_, ri_)])

    return k(head8, col8, ones, z16)


def _sc_kg(ent_tbl, eidx, w_r, z32):
    r = NEP // NTILE

    @functools.partial(
        pl.kernel, mesh=_sc_mesh(), compiler_params=_SC_PARAMS,
        out_type=jax.ShapeDtypeStruct((NSC, NEP, H), f32),
        scratch_types=[pltpu.VMEM((2, 3, GE, CH), i32),
                       pltpu.VMEM((2, GEDGE, H), f32),
                       pltpu.VMEM((GEDGE, H), f32),
                       pltpu.VMEM_SHARED((NEP, H), f32),
                       pltpu.VMEM_SHARED((NR, H), f32),
                       pltpu.SemaphoreType.DMA,
                       pltpu.SemaphoreType.DMA])
    def k(ent_h, eidx_h, w_h, z_h, out_h,
          cidx, ent_v, w_v, acc, w_sp, gsem, ssem):
        c = lax.axis_index("c")
        s = lax.axis_index("s")
        pltpu.sync_copy(z_h.at[pl.ds(s * r, r)], acc.at[pl.ds(s * r, r)])

        @pl.when(s == 0)
        def _():
            pltpu.sync_copy(w_h.at[c], w_v.at[pl.ds(0, NR)])
            pltpu.sync_copy(w_v.at[pl.ds(0, NR)], w_sp)

        plsc.subcore_barrier()

        def load_and_gather(g, h):
            pltpu.sync_copy(eidx_h.at[c, s * NG_E + g], cidx.at[h])
            for j in range(GE):
                pltpu.async_copy(ent_h.at[cidx.at[h, 0, j]],
                                 ent_v.at[h, pl.ds(j * CH, CH)], gsem)

        load_and_gather(0, 0)

        @pl.loop(0, NG_E)
        def _(g):
            p = lax.rem(g, 2)
            q = 1 - p

            @pl.when(g + 1 < NG_E)
            def _():
                load_and_gather(g + 1, q)

            for j in range(GE):
                pltpu.make_async_copy(ent_h.at[cidx.at[p, 0, j]],
                                      ent_v.at[p, pl.ds(j * CH, CH)],
                                      gsem).wait()
            for j in range(GE):
                pltpu.sync_copy(w_sp.at[cidx.at[p, 1, j]],
                                w_v.at[pl.ds(j * CH, CH)])
            eb = ent_v.at[p]
            wb = w_v

            @pl.loop(0, GEDGE, step=4)
            def _(j):
                for dj in range(4):
                    for h0 in (0, 16):
                        eb[j + dj, pl.ds(h0, 16)] = (
                            eb[j + dj, pl.ds(h0, 16)]
                            * wb[j + dj, pl.ds(h0, 16)])

            shs = [pltpu.async_copy(ent_v.at[p, pl.ds(j * CH, CH)],
                                    acc.at[cidx.at[p, 2, j]], ssem, add=True)
                   for j in range(GE)]
            for sh in shs:
                sh.wait()

        plsc.subcore_barrier()
        pltpu.sync_copy(acc.at[pl.ds(s * r, r)], out_h.at[c, pl.ds(s * r, r)])

    return k(ent_tbl, eidx, w_r, z32)


def _sc_gs(tbl, gidx8, sidx8, acc_rows, z32):
    r = acc_rows // NTILE

    @functools.partial(
        pl.kernel, mesh=_sc_mesh(), compiler_params=_SC_PARAMS,
        out_type=jax.ShapeDtypeStruct((NSC, acc_rows, H), f32),
        scratch_types=[pltpu.VMEM((2, GS, CH), i32),
                       pltpu.VMEM((2, GS, CH), i32),
                       pltpu.VMEM((2, GROW, H), f32),
                       pltpu.VMEM_SHARED((acc_rows, H), f32),
                       pltpu.SemaphoreType.DMA,
                       pltpu.SemaphoreType.DMA])
    def k(tbl_h, g_h, s_h, z_h, out_h, gi, si, rows_v, acc, gsem, ssem):
        c = lax.axis_index("c")
        s = lax.axis_index("s")
        pltpu.sync_copy(z_h.at[pl.ds(s * r, r)], acc.at[pl.ds(s * r, r)])
        plsc.subcore_barrier()

        def load_and_gather(g, h):
            gb = s * NG_M + g
            pltpu.sync_copy(g_h.at[c, gb], gi.at[h])
            pltpu.sync_copy(s_h.at[gb], si.at[h])
            for j in range(GS):
                pltpu.async_copy(tbl_h.at[gi.at[h, j]],
                                 rows_v.at[h, pl.ds(j * CH, CH)], gsem)

        def scat(h, wait):
            if wait:
                return
            hs = [pltpu.async_copy(rows_v.at[h, pl.ds(j * CH, CH)],
                                   acc.at[si.at[h, j]], ssem, add=True)
                  for j in range(GS)]
            for r_ in hs:
                r_.wait()

        load_and_gather(0, 0)

        @pl.loop(0, NG_M)
        def _(g):
            p = lax.rem(g, 2)
            q = 1 - p

            @pl.when(g >= 1)
            def _():
                scat(q, wait=True)

            @pl.when(g + 1 < NG_M)
            def _():
                load_and_gather(g + 1, q)

            for j in range(GS):
                pltpu.make_async_copy(tbl_h.at[gi.at[p, j]],
                                      rows_v.at[p, pl.ds(j * CH, CH)],
                                      gsem).wait()
            scat(p, wait=False)

        scat(lax.rem(NG_M - 1, 2), wait=True)
        plsc.subcore_barrier()
        pltpu.sync_copy(acc.at[pl.ds(s * r, r)], out_h.at[c, pl.ds(s * r, r)])

    return k(tbl, gidx8, sidx8, z32)


def _nrm(x):
    n = jnp.sqrt(jnp.sum(x * x, axis=1, keepdims=True))
    return x / jnp.maximum(n, 1e-12)


BR = 2000
NB_I = NI // BR


def _tc_fuse(kg_sum, cnt_e, int_sum, cnt_i, g1, g2, eres, kres, ires):
    nb = NE // BR

    def body(kg_ref, ce_ref, it_ref, ci_ref, g1_ref, g2_ref, er_ref, kr_ref,
             ir_ref, ent_ref, fus_ref, ero_ref, kro_ref, iro_ref):
        j = pl.program_id(0)
        kg = jnp.concatenate([kg_ref[0], kg_ref[1]], axis=1)
        ce = jnp.maximum(ce_ref[:, 0:1], 1.0)
        kg_agg = kg / ce
        it = jnp.concatenate([it_ref[0], it_ref[1]], axis=1)
        ci = jnp.maximum(ci_ref[:, 0:1], 1.0)
        it_agg = it / ci
        dn = (((1,), (1,)), ((), ()))
        z = (lax.dot_general(kg_agg, g1_ref[...], dn, precision=lax.Precision.HIGHEST)
             + lax.dot_general(it_agg, g2_ref[...], dn, precision=lax.Precision.HIGHEST))
        gi = jax.nn.sigmoid(z)
        fusion = gi * kg_agg + (1.0 - gi) * it_agg
        fus_val = jnp.where(j < NB_I, fusion, kg_agg)
        ent_n = _nrm(fus_val)
        ent_ref[0] = ent_n[:, :H]
        ent_ref[1] = ent_n[:, H:]
        fus_ref[0] = fus_val[:, :H]
        fus_ref[1] = fus_val[:, H:]
        ero_ref[...] = er_ref[...] + ent_n
        kro_ref[...] = kr_ref[...] + _nrm(kg_agg)
        iro_ref[...] = ir_ref[...] + _nrm(it_agg)

    item_blk3 = lambda j: (0, jnp.minimum(j, NB_I - 1), 0)
    item_blk2 = lambda j: (jnp.minimum(j, NB_I - 1), 0)
    return pl.pallas_call(
        body,
        grid=(nb,),
        in_specs=[
            pl.BlockSpec((NSC, BR, H), lambda j: (0, j, 0)),
            pl.BlockSpec((BR, 16), lambda j: (j, 0)),
            pl.BlockSpec((NSC, BR, H), item_blk3),
            pl.BlockSpec((BR, 16), item_blk2),
            pl.BlockSpec((C, C), lambda j: (0, 0)),
            pl.BlockSpec((C, C), lambda j: (0, 0)),
            pl.BlockSpec((BR, C), lambda j: (j, 0)),
            pl.BlockSpec((BR, C), lambda j: (j, 0)),
            pl.BlockSpec((BR, C), lambda j: (j, 0)),
        ],
        out_specs=[
            pl.BlockSpec((NSC, BR, H), lambda j: (0, j, 0)),
            pl.BlockSpec((NSC, BR, H), lambda j: (0, j, 0)),
            pl.BlockSpec((BR, C), lambda j: (j, 0)),
            pl.BlockSpec((BR, C), lambda j: (j, 0)),
            pl.BlockSpec((BR, C), lambda j: (j, 0)),
        ],
        out_shape=[
            jax.ShapeDtypeStruct((NSC, NE, H), f32),
            jax.ShapeDtypeStruct((NSC, NE, H), f32),
            jax.ShapeDtypeStruct((NE, C), f32),
            jax.ShapeDtypeStruct((NE, C), f32),
            jax.ShapeDtypeStruct((NE, C), f32),
        ],
    )(kg_sum, cnt_e, int_sum, cnt_i, g1, g2, eres, kres, ires)


def _tc_user(usum, ures):
    nb = NU // BR

    def body(us_ref, ur_ref, un_ref, uro_ref):
        us = jnp.concatenate([us_ref[0], us_ref[1]], axis=1)
        un = _nrm(us)
        un_ref[0] = un[:, :H]
        un_ref[1] = un[:, H:]
        uro_ref[...] = ur_ref[...] + un

    return pl.pallas_call(
        body,
        grid=(nb,),
        in_specs=[
            pl.BlockSpec((NSC, BR, H), lambda j: (0, j, 0)),
            pl.BlockSpec((BR, C), lambda j: (j, 0)),
        ],
        out_specs=[
            pl.BlockSpec((NSC, BR, H), lambda j: (0, j, 0)),
            pl.BlockSpec((BR, C), lambda j: (j, 0)),
        ],
        out_shape=[
            jax.ShapeDtypeStruct((NSC, NU, H), f32),
            jax.ShapeDtypeStruct((NU, C), f32),
        ],
    )(usum, ures)


def kernel(user_emb, entity_emb, edge_index, edge_type, interact_mat,
           weight, gate1_w, gate2_w):
    head = edge_index[0].astype(i32)
    tail = edge_index[1].astype(i32)
    et = edge_type.astype(i32)
    mrow = interact_mat[0].astype(i32)
    mcol = interact_mat[1].astype(i32)

    epad = EP - E
    eord = jnp.arange(epad, dtype=i32)
    tail_p = jnp.concatenate([tail, eord % 512])
    head_p = jnp.concatenate([head, NE + eord % (NEP - NE)])
    et_p = jnp.concatenate([et, jnp.zeros((epad,), i32)])

    mpad = NNZP - NNZ
    mord = jnp.arange(mpad, dtype=i32)
    mrow_g = jnp.concatenate([mrow, mord % 512])
    mcol_s = jnp.concatenate([mcol, NI + mord % (NIP - NI)])
    mcol_g = jnp.concatenate([mcol, mord % 512])
    mrow_s = jnp.concatenate([mrow, NU + mord % (NUP - NU)])

    et3 = et_p.reshape(-1, GE, CH)
    hd3 = head_p.reshape(-1, GE, CH)
    eidx = jnp.stack([
        jnp.stack([(tail_p + c * NE).reshape(-1, GE, CH), et3, hd3], axis=1)
        for c in range(NSC)])
    head8 = head_p.reshape(-1, GS, CH)
    mrow_g8 = jnp.stack([mrow_g, mrow_g + NU]).reshape(NSC, -1, GS, CH)
    mcol_s8 = mcol_s.reshape(-1, GS, CH)
    mcol_g8 = jnp.stack([mcol_g, mcol_g + NE]).reshape(NSC, -1, GS, CH)
    mrow_s8 = mrow_s.reshape(-1, GS, CH)

    z16 = jnp.zeros((NEP, 16), f32)
    z32 = jnp.zeros((NEP, H), f32)
    ones = jnp.ones((CH, 16), f32)

    w_r = weight.reshape(NR, NSC, H).transpose(1, 0, 2)
    ent_tbl = entity_emb.reshape(NE, NSC, H).transpose(1, 0, 2).reshape(NSC * NE, H)
    usr_tbl = user_emb.reshape(NU, NSC, H).transpose(1, 0, 2).reshape(NSC * NU, H)

    cnt_e, cnt_i = _sc_counts(head8, mcol_s8, ones, z16)

    eres = entity_emb
    ures = user_emb
    kres = jnp.zeros((NE, C), f32)
    ires = jnp.zeros((NE, C), f32)

    for i in range(HOPS):
        kg_sum = _sc_kg(ent_tbl, eidx, w_r, z32)
        int_sum = _sc_gs(usr_tbl, mrow_g8, mcol_s8, NIP, z32)
        ent_new, fus, eres, kres, ires = _tc_fuse(
            kg_sum, cnt_e, int_sum, cnt_i, gate1_w[i], gate2_w[i],
            eres, kres, ires)
        usum = _sc_gs(fus.reshape(NSC * NE, H), mcol_g8, mrow_s8, NUP, z32)
        usr_new, ures = _tc_user(usum, ures)
        ent_tbl = ent_new.reshape(NSC * NE, H)
        usr_tbl = usr_new.reshape(NSC * NU, H)

    return (eres, ures, kres[:NI], ires[:NI])


# speedup vs baseline: 6.4001x; 1.0834x over previous
import functools

import jax
import jax.numpy as jnp
from jax import lax
from jax.experimental import pallas as pl
from jax.experimental.pallas import tpu as pltpu
from jax.experimental.pallas import tpu_sc as plsc

NU, NI, NE, NR = 20000, 30000, 50000, 16
E, NNZ, C, HOPS = 800000, 500000, 64, 2
H = C // 2
CH = 128
NSC, NTILE, NW = 2, 16, 32

GE = 2
GEDGE = GE * CH
NG_E = 196
EP = NTILE * NG_E * GEDGE
GS = 8
GROW = GS * CH
NG_M = 31
NNZP = NTILE * NG_M * GROW
NG_C = 49

NEP = 50048
NIP = 30080
NUP = 20096

f32 = jnp.float32
i32 = jnp.int32

_SC_PARAMS = pltpu.CompilerParams(use_tc_tiling_on_sc=False)


@functools.lru_cache(maxsize=None)
def _sc_mesh():
    return plsc.VectorSubcoreMesh(core_axis_name="c", subcore_axis_name="s",
                                  num_cores=NSC, num_subcores=NTILE)


def _hist_pipeline(idx_h, ng, ones_v, cidx, acc, ssem, s):
    def load(g, h):
        pltpu.sync_copy(idx_h.at[s * ng + g], cidx.at[h])

    def scat(h, wait):
        if wait:
            return
        hs = [pltpu.async_copy(ones_v, acc.at[cidx.at[h, j]], ssem, add=True)
              for j in range(GS)]
        for r_ in hs:
            r_.wait()

    load(0, 0)

    @pl.loop(0, ng)
    def _(g):
        p = lax.rem(g, 2)
        q = 1 - p

        @pl.when(g >= 1)
        def _():
            scat(q, wait=True)

        @pl.when(g + 1 < ng)
        def _():
            load(g + 1, q)

        scat(p, wait=False)

    scat(lax.rem(ng - 1, 2), wait=True)


def _sc_counts(head8, col8, ones, z16):
    re_, ri_ = NEP // NTILE, NIP // NTILE

    @functools.partial(
        pl.kernel, mesh=_sc_mesh(), compiler_params=_SC_PARAMS,
        out_type=(jax.ShapeDtypeStruct((NEP, 16), f32),
                  jax.ShapeDtypeStruct((NIP, 16), f32)),
        scratch_types=[pltpu.VMEM((CH, 16), f32),
                       pltpu.VMEM((2, GS, CH), i32),
                       pltpu.VMEM_SHARED((NEP, 16), f32),
                       pltpu.SemaphoreType.DMA])
    def k(head_h, col_h, ones_h, z_h, ce_h, ci_h, ones_v, cidx, acc, ssem):
        c = lax.axis_index("c")
        s = lax.axis_index("s")
        pltpu.sync_copy(ones_h, ones_v)

        @pl.when(c == 0)
        def _():
            pltpu.sync_copy(z_h.at[pl.ds(s * re_, re_)],
                            acc.at[pl.ds(s * re_, re_)])
            plsc.subcore_barrier()
            _hist_pipeline(head_h, NG_C, ones_v, cidx, acc, ssem, s)
            plsc.subcore_barrier()
            pltpu.sync_copy(acc.at[pl.ds(s * re_, re_)],
                            ce_h.at[pl.ds(s * re_, re_)])

        @pl.when(c == 1)
        def _():
            pltpu.sync_copy(z_h.at[pl.ds(s * ri_, ri_)],
                            acc.at[pl.ds(s * ri_, ri_)])
            plsc.subcore_barrier()
            _hist_pipeline(col_h, NG_M, ones_v, cidx, acc, ssem, s)
            plsc.subcore_barrier()
            pltpu.sync_copy(acc.at[pl.ds(s * ri_, ri_)],
                            ci_h.at[pl.ds(s * ri_, ri_)])

    return k(head8, col8, ones, z16)


def _sc_kg(ent_tbl, eidx, w_r, z32):
    r = NEP // NTILE

    @functools.partial(
        pl.kernel, mesh=_sc_mesh(), compiler_params=_SC_PARAMS,
        out_type=jax.ShapeDtypeStruct((NSC, NEP, H), f32),
        scratch_types=[pltpu.VMEM((2, 3, GE, CH), i32),
                       pltpu.VMEM((2, GEDGE, H), f32),
                       pltpu.VMEM((GEDGE, H), f32),
                       pltpu.VMEM_SHARED((NEP, H), f32),
                       pltpu.VMEM_SHARED((NR, H), f32),
                       pltpu.SemaphoreType.DMA,
                       pltpu.SemaphoreType.DMA])
    def k(ent_h, eidx_h, w_h, z_h, out_h,
          cidx, ent_v, w_v, acc, w_sp, gsem, ssem):
        c = lax.axis_index("c")
        s = lax.axis_index("s")
        pltpu.sync_copy(z_h.at[pl.ds(s * r, r)], acc.at[pl.ds(s * r, r)])

        @pl.when(s == 0)
        def _():
            pltpu.sync_copy(w_h.at[c], w_v.at[pl.ds(0, NR)])
            pltpu.sync_copy(w_v.at[pl.ds(0, NR)], w_sp)

        plsc.subcore_barrier()

        def load_and_gather(g, h):
            pltpu.sync_copy(eidx_h.at[c, s * NG_E + g], cidx.at[h])
            for j in range(GE):
                pltpu.async_copy(ent_h.at[cidx.at[h, 0, j]],
                                 ent_v.at[h, pl.ds(j * CH, CH)], gsem)

        load_and_gather(0, 0)

        @pl.loop(0, NG_E)
        def _(g):
            p = lax.rem(g, 2)
            q = 1 - p

            whs = [pltpu.async_copy(w_sp.at[cidx.at[p, 1, j]],
                                    w_v.at[pl.ds(j * CH, CH)], ssem)
                   for j in range(GE)]

            @pl.when(g + 1 < NG_E)
            def _():
                load_and_gather(g + 1, q)

            for j in range(GE):
                pltpu.make_async_copy(ent_h.at[cidx.at[p, 0, j]],
                                      ent_v.at[p, pl.ds(j * CH, CH)],
                                      gsem).wait()
            for wh in whs:
                wh.wait()
            eb = ent_v.at[p]
            wb = w_v

            @pl.loop(0, GEDGE, step=8)
            def _(j):
                for dj in range(8):
                    for h0 in (0, 16):
                        eb[j + dj, pl.ds(h0, 16)] = (
                            eb[j + dj, pl.ds(h0, 16)]
                            * wb[j + dj, pl.ds(h0, 16)])

            shs = [pltpu.async_copy(ent_v.at[p, pl.ds(j * CH, CH)],
                                    acc.at[cidx.at[p, 2, j]], ssem, add=True)
                   for j in range(GE)]
            for sh in shs:
                sh.wait()

        plsc.subcore_barrier()
        pltpu.sync_copy(acc.at[pl.ds(s * r, r)], out_h.at[c, pl.ds(s * r, r)])

    return k(ent_tbl, eidx, w_r, z32)


def _sc_gs(tbl, gidx8, sidx8, acc_rows, z32):
    r = acc_rows // NTILE

    @functools.partial(
        pl.kernel, mesh=_sc_mesh(), compiler_params=_SC_PARAMS,
        out_type=jax.ShapeDtypeStruct((NSC, acc_rows, H), f32),
        scratch_types=[pltpu.VMEM((2, GS, CH), i32),
                       pltpu.VMEM((2, GS, CH), i32),
                       pltpu.VMEM((2, GROW, H), f32),
                       pltpu.VMEM_SHARED((acc_rows, H), f32),
                       pltpu.SemaphoreType.DMA,
                       pltpu.SemaphoreType.DMA])
    def k(tbl_h, g_h, s_h, z_h, out_h, gi, si, rows_v, acc, gsem, ssem):
        c = lax.axis_index("c")
        s = lax.axis_index("s")
        pltpu.sync_copy(z_h.at[pl.ds(s * r, r)], acc.at[pl.ds(s * r, r)])
        plsc.subcore_barrier()

        def load_and_gather(g, h):
            gb = s * NG_M + g
            pltpu.sync_copy(g_h.at[c, gb], gi.at[h])
            pltpu.sync_copy(s_h.at[gb], si.at[h])
            for j in range(GS):
                pltpu.async_copy(tbl_h.at[gi.at[h, j]],
                                 rows_v.at[h, pl.ds(j * CH, CH)], gsem)

        def scat(h, wait):
            if wait:
                return
            hs = [pltpu.async_copy(rows_v.at[h, pl.ds(j * CH, CH)],
                                   acc.at[si.at[h, j]], ssem, add=True)
                  for j in range(GS)]
            for r_ in hs:
                r_.wait()

        load_and_gather(0, 0)

        @pl.loop(0, NG_M)
        def _(g):
            p = lax.rem(g, 2)
            q = 1 - p

            @pl.when(g >= 1)
            def _():
                scat(q, wait=True)

            @pl.when(g + 1 < NG_M)
            def _():
                load_and_gather(g + 1, q)

            for j in range(GS):
                pltpu.make_async_copy(tbl_h.at[gi.at[p, j]],
                                      rows_v.at[p, pl.ds(j * CH, CH)],
                                      gsem).wait()
            scat(p, wait=False)

        scat(lax.rem(NG_M - 1, 2), wait=True)
        plsc.subcore_barrier()
        pltpu.sync_copy(acc.at[pl.ds(s * r, r)], out_h.at[c, pl.ds(s * r, r)])

    return k(tbl, gidx8, sidx8, z32)


def _nrm(x):
    n = jnp.sqrt(jnp.sum(x * x, axis=1, keepdims=True))
    return x / jnp.maximum(n, 1e-12)


BR = 2000
NB_I = NI // BR


def _tc_fuse(kg_sum, cnt_e, int_sum, cnt_i, g1, g2, eres, kres, ires):
    nb = NE // BR

    def body(kg_ref, ce_ref, it_ref, ci_ref, g1_ref, g2_ref, er_ref, kr_ref,
             ir_ref, ent_ref, fus_ref, ero_ref, kro_ref, iro_ref):
        j = pl.program_id(0)
        kg = jnp.concatenate([kg_ref[0], kg_ref[1]], axis=1)
        ce = jnp.maximum(ce_ref[:, 0:1], 1.0)
        kg_agg = kg / ce
        it = jnp.concatenate([it_ref[0], it_ref[1]], axis=1)
        ci = jnp.maximum(ci_ref[:, 0:1], 1.0)
        it_agg = it / ci
        dn = (((1,), (1,)), ((), ()))
        z = (lax.dot_general(kg_agg, g1_ref[...], dn, precision=lax.Precision.HIGHEST)
             + lax.dot_general(it_agg, g2_ref[...], dn, precision=lax.Precision.HIGHEST))
        gi = jax.nn.sigmoid(z)
        fusion = gi * kg_agg + (1.0 - gi) * it_agg
        fus_val = jnp.where(j < NB_I, fusion, kg_agg)
        ent_n = _nrm(fus_val)
        ent_ref[0] = ent_n[:, :H]
        ent_ref[1] = ent_n[:, H:]
        fus_ref[0] = fus_val[:, :H]
        fus_ref[1] = fus_val[:, H:]
        ero_ref[...] = er_ref[...] + ent_n
        kro_ref[...] = kr_ref[...] + _nrm(kg_agg)
        iro_ref[...] = ir_ref[...] + _nrm(it_agg)

    item_blk3 = lambda j: (0, jnp.minimum(j, NB_I - 1), 0)
    item_blk2 = lambda j: (jnp.minimum(j, NB_I - 1), 0)
    return pl.pallas_call(
        body,
        grid=(nb,),
        in_specs=[
            pl.BlockSpec((NSC, BR, H), lambda j: (0, j, 0)),
            pl.BlockSpec((BR, 16), lambda j: (j, 0)),
            pl.BlockSpec((NSC, BR, H), item_blk3),
            pl.BlockSpec((BR, 16), item_blk2),
            pl.BlockSpec((C, C), lambda j: (0, 0)),
            pl.BlockSpec((C, C), lambda j: (0, 0)),
            pl.BlockSpec((BR, C), lambda j: (j, 0)),
            pl.BlockSpec((BR, C), lambda j: (j, 0)),
            pl.BlockSpec((BR, C), lambda j: (j, 0)),
        ],
        out_specs=[
            pl.BlockSpec((NSC, BR, H), lambda j: (0, j, 0)),
            pl.BlockSpec((NSC, BR, H), lambda j: (0, j, 0)),
            pl.BlockSpec((BR, C), lambda j: (j, 0)),
            pl.BlockSpec((BR, C), lambda j: (j, 0)),
            pl.BlockSpec((BR, C), lambda j: (j, 0)),
        ],
        out_shape=[
            jax.ShapeDtypeStruct((NSC, NE, H), f32),
            jax.ShapeDtypeStruct((NSC, NE, H), f32),
            jax.ShapeDtypeStruct((NE, C), f32),
            jax.ShapeDtypeStruct((NE, C), f32),
            jax.ShapeDtypeStruct((NE, C), f32),
        ],
    )(kg_sum, cnt_e, int_sum, cnt_i, g1, g2, eres, kres, ires)


def _tc_user(usum, ures):
    nb = NU // BR

    def body(us_ref, ur_ref, un_ref, uro_ref):
        us = jnp.concatenate([us_ref[0], us_ref[1]], axis=1)
        un = _nrm(us)
        un_ref[0] = un[:, :H]
        un_ref[1] = un[:, H:]
        uro_ref[...] = ur_ref[...] + un

    return pl.pallas_call(
        body,
        grid=(nb,),
        in_specs=[
            pl.BlockSpec((NSC, BR, H), lambda j: (0, j, 0)),
            pl.BlockSpec((BR, C), lambda j: (j, 0)),
        ],
        out_specs=[
            pl.BlockSpec((NSC, BR, H), lambda j: (0, j, 0)),
            pl.BlockSpec((BR, C), lambda j: (j, 0)),
        ],
        out_shape=[
            jax.ShapeDtypeStruct((NSC, NU, H), f32),
            jax.ShapeDtypeStruct((NU, C), f32),
        ],
    )(usum, ures)


def kernel(user_emb, entity_emb, edge_index, edge_type, interact_mat,
           weight, gate1_w, gate2_w):
    head = edge_index[0].astype(i32)
    tail = edge_index[1].astype(i32)
    et = edge_type.astype(i32)
    mrow = interact_mat[0].astype(i32)
    mcol = interact_mat[1].astype(i32)

    epad = EP - E
    eord = jnp.arange(epad, dtype=i32)
    tail_p = jnp.concatenate([tail, eord % 512])
    head_p = jnp.concatenate([head, NE + eord % (NEP - NE)])
    et_p = jnp.concatenate([et, jnp.zeros((epad,), i32)])

    mpad = NNZP - NNZ
    mord = jnp.arange(mpad, dtype=i32)
    mrow_g = jnp.concatenate([mrow, mord % 512])
    mcol_s = jnp.concatenate([mcol, NI + mord % (NIP - NI)])
    mcol_g = jnp.concatenate([mcol, mord % 512])
    mrow_s = jnp.concatenate([mrow, NU + mord % (NUP - NU)])

    et3 = et_p.reshape(-1, GE, CH)
    hd3 = head_p.reshape(-1, GE, CH)
    eidx = jnp.stack([
        jnp.stack([(tail_p + c * NE).reshape(-1, GE, CH), et3, hd3], axis=1)
        for c in range(NSC)])
    head8 = head_p.reshape(-1, GS, CH)
    mrow_g8 = jnp.stack([mrow_g, mrow_g + NU]).reshape(NSC, -1, GS, CH)
    mcol_s8 = mcol_s.reshape(-1, GS, CH)
    mcol_g8 = jnp.stack([mcol_g, mcol_g + NE]).reshape(NSC, -1, GS, CH)
    mrow_s8 = mrow_s.reshape(-1, GS, CH)

    z16 = jnp.zeros((NEP, 16), f32)
    z32 = jnp.zeros((NEP, H), f32)
    ones = jnp.ones((CH, 16), f32)

    w_r = weight.reshape(NR, NSC, H).transpose(1, 0, 2)
    ent_tbl = entity_emb.reshape(NE, NSC, H).transpose(1, 0, 2).reshape(NSC * NE, H)
    usr_tbl = user_emb.reshape(NU, NSC, H).transpose(1, 0, 2).reshape(NSC * NU, H)

    cnt_e, cnt_i = _sc_counts(head8, mcol_s8, ones, z16)

    eres = entity_emb
    ures = user_emb
    kres = jnp.zeros((NE, C), f32)
    ires = jnp.zeros((NE, C), f32)

    for i in range(HOPS):
        kg_sum = _sc_kg(ent_tbl, eidx, w_r, z32)
        int_sum = _sc_gs(usr_tbl, mrow_g8, mcol_s8, NIP, z32)
        ent_new, fus, eres, kres, ires = _tc_fuse(
            kg_sum, cnt_e, int_sum, cnt_i, gate1_w[i], gate2_w[i],
            eres, kres, ires)
        usum = _sc_gs(fus.reshape(NSC * NE, H), mcol_g8, mrow_s8, NUP, z32)
        usr_new, ures = _tc_user(usum, ures)
        ent_tbl = ent_new.reshape(NSC * NE, H)
        usr_tbl = usr_new.reshape(NSC * NU, H)

    return (eres, ures, kres[:NI], ires[:NI])
